# Initial kernel scaffold; baseline (speedup 1.0000x reference)
import jax
import jax.numpy as jnp
from jax.experimental import pallas as pl


def kernel(x, pk_embeddings, pk_predictions, edge_index, gate_w, gate_b, conv_w0, conv_b0, conv_w1, conv_b1, conv_w2, conv_b2, ln_g0, ln_b0, ln_g1, ln_b1, ln_g2, ln_b2, pre_w, pre_b, q_weights, post_w, post_b, res_w1, res_b1, res_w2, res_b2, res_alpha):
    raise NotImplementedError("write your pallas kernel here")


# speedup vs baseline: 9.9389x; 9.9389x over previous
import functools
import numpy as np
import jax
import jax.numpy as jnp
from jax import lax
from jax.experimental import pallas as pl
from jax.experimental.pallas import tpu as pltpu
from jax.experimental.pallas import tpu_sc as plsc

N = 50000
NPAD = 51200
ATOT = NPAD + 128
E = 800000
EPAD = 819200
H = 64
HH = 16
NS = 4
ROW = 128
NROWS = EPAD // ROW
GROUP = 8
ZCH = 128
BLK = 1000
GRID = N // BLK

_mesh = plsc.VectorSubcoreMesh(core_axis_name="c", subcore_axis_name="s")
f32 = jnp.float32
_sc_params = pltpu.CompilerParams(use_tc_tiling_on_sc=False)


@functools.partial(
    pl.kernel,
    out_type=[jax.ShapeDtypeStruct((NPAD, 16), f32),
              jax.ShapeDtypeStruct((NPAD, 16), f32)],
    mesh=_mesh,
    scratch_types=[
        pltpu.VMEM((ROW, 16), f32),
        pltpu.VMEM((ZCH, 16), f32),
        pltpu.VMEM((GROUP, ROW), jnp.int32),
        pltpu.VMEM_SHARED((ATOT, 16), f32),
    ],
    compiler_params=_sc_params,
)
def _deg_kernel(dst2d, deg0_out, deg1_out, ones_v, zero_v, idx_d, acc):
    c = lax.axis_index("c")
    s = lax.axis_index("s")
    npt = NPAD // 16

    def fill(i, _):
        ones_v[i, :] = jnp.ones((16,), f32)
        return 0
    lax.fori_loop(0, ROW, fill, 0)

    def fillz(i, _):
        zero_v[i, :] = jnp.zeros((16,), f32)
        return 0
    lax.fori_loop(0, ZCH, fillz, 0)

    def zero(i, _):
        pltpu.sync_copy(zero_v, acc.at[pl.ds(s * npt + i * ZCH, ZCH)])
        return 0
    lax.fori_loop(0, npt // ZCH, zero, 0)
    plsc.subcore_barrier()

    rows_per_tile = NROWS // 32
    base = (c * 16 + s) * rows_per_tile

    def grp(g, _):
        pltpu.sync_copy(dst2d.at[pl.ds(base + g * GROUP, GROUP)], idx_d)
        for j in range(GROUP):
            pltpu.sync_copy(ones_v, acc.at[idx_d.at[j]], add=True)
        return 0
    lax.fori_loop(0, rows_per_tile // GROUP, grp, 0)
    plsc.subcore_barrier()

    @pl.when(c == 0)
    def _():
        pltpu.sync_copy(acc.at[pl.ds(s * npt, npt)],
                        deg0_out.at[pl.ds(s * npt, npt)])

    @pl.when(c == 1)
    def _():
        pltpu.sync_copy(acc.at[pl.ds(s * npt, npt)],
                        deg1_out.at[pl.ds(s * npt, npt)])


@functools.partial(
    pl.kernel,
    out_type=[jax.ShapeDtypeStruct((NPAD, HH), f32) for _ in range(NS)],
    mesh=_mesh,
    scratch_types=[
        pltpu.VMEM((GROUP, ROW), jnp.int32),
        pltpu.VMEM((GROUP, ROW), jnp.int32),
        pltpu.VMEM((GROUP, ROW, HH), f32),
        pltpu.VMEM((ZCH, HH), f32),
        pltpu.VMEM_SHARED((ATOT, HH), f32),
        pltpu.SemaphoreType.DMA,
    ],
    compiler_params=_sc_params,
)
def _scatter_kernel(src2d, dst2d, y0, y1, y2, y3, z0, z1, z2, z3,
                    idx_s, idx_d, rows, zero_v, acc, sem):
    c = lax.axis_index("c")
    s = lax.axis_index("s")
    npt = NPAD // 16

    def fillz(i, _):
        zero_v[i, :] = jnp.zeros((16,), f32)
        return 0
    lax.fori_loop(0, ZCH, fillz, 0)

    rows_per_tile = NROWS // 16
    base = s * rows_per_tile

    def phase(y_ref, z_ref):
        def zero(i, _):
            pltpu.sync_copy(zero_v, acc.at[pl.ds(s * npt + i * ZCH, ZCH)])
            return 0
        lax.fori_loop(0, npt // ZCH, zero, 0)
        plsc.subcore_barrier()

        def grp(g, _):
            r0 = base + g * GROUP
            pltpu.sync_copy(src2d.at[pl.ds(r0, GROUP)], idx_s)
            pltpu.sync_copy(dst2d.at[pl.ds(r0, GROUP)], idx_d)
            descs = [pltpu.async_copy(y_ref.at[idx_s.at[j]], rows.at[j], sem)
                     for j in range(GROUP)]
            for d in descs:
                d.wait()
            for j in range(GROUP):
                pltpu.sync_copy(rows.at[j], acc.at[idx_d.at[j]], add=True)
            return 0
        lax.fori_loop(0, rows_per_tile // GROUP, grp, 0)
        plsc.subcore_barrier()
        pltpu.sync_copy(acc.at[pl.ds(s * npt, npt)],
                        z_ref.at[pl.ds(s * npt, npt)])
        plsc.subcore_barrier()

    ys = [y0, y1, y2, y3]
    zs = [z0, z1, z2, z3]
    for q in range(2):
        @pl.when(c == 0)
        def _(q=q):
            phase(ys[q], zs[q])

        @pl.when(c == 1)
        def _(q=q):
            phase(ys[2 + q], zs[2 + q])


def _stage_a_body(comb, deg0, deg1, gate_wT, gate_b, w0, rw1T, rb1, rw2T,
                  rb2a, gate_o, y0_o, y1_o, y2_o, y3_o, res_o, dinv_o):
    x = comb[...]
    deg = deg0[:, 0:1] + deg1[:, 0:1] + 1.0
    dinv = lax.rsqrt(deg)
    g = jax.nn.sigmoid(jnp.dot(x, gate_wT[...],
                               preferred_element_type=f32) + gate_b[...])
    y = dinv * jnp.dot(x, w0[...], preferred_element_type=f32)
    r = jax.nn.relu(jnp.dot(x, rw1T[...], preferred_element_type=f32)
                    + rb1[...])
    r = jnp.dot(r, rw2T[...], preferred_element_type=f32) + rb2a[0, 0:1]
    gate_o[...] = g
    y0_o[...] = y[:, 0:16]
    y1_o[...] = y[:, 16:32]
    y2_o[...] = y[:, 32:48]
    y3_o[...] = y[:, 48:64]
    res_o[...] = r * rb2a[0, 1:2]
    dinv_o[...] = dinv


def _rowspec(k):
    return pl.BlockSpec((BLK, k), lambda i: (i, 0))


def _wspec(r, k):
    return pl.BlockSpec((r, k), lambda i: (0, 0))


def _stage_a(comb, deg0, deg1, gate_wT, gate_b, w0, rw1T, rb1, rw2T, rb2a):
    return pl.pallas_call(
        _stage_a_body,
        grid=(GRID,),
        in_specs=[_rowspec(H), _rowspec(16), _rowspec(16), _wspec(H, H),
                  _wspec(1, H), _wspec(H, H), _wspec(H, 32), _wspec(1, 32),
                  _wspec(32, 1), _wspec(1, 2)],
        out_specs=[_rowspec(H)] + [_rowspec(HH)] * NS
        + [_rowspec(1), _rowspec(1)],
        out_shape=[jax.ShapeDtypeStruct((N, H), f32)]
        + [jax.ShapeDtypeStruct((N, HH), f32) for _ in range(NS)]
        + [jax.ShapeDtypeStruct((N, 1), f32),
           jax.ShapeDtypeStruct((N, 1), f32)],
    )(comb, deg0, deg1, gate_wT, gate_b, w0, rw1T, rb1, rw2T, rb2a)


def _stage_b_body(mode, z0, z1, z2, z3, y0, y1, y2, y3, dinv, aux,
                  b_c, ln_g, ln_b, w_n, h_o, y0_o, y1_o, y2_o, y3_o):
    di = dinv[...]
    t = jnp.concatenate([z0[...] + y0[...], z1[...] + y1[...],
                         z2[...] + y2[...], z3[...] + y3[...]], axis=-1)
    t = di * t + b_c[...]
    m = jnp.mean(t, -1, keepdims=True)
    v = jnp.mean(t * t, -1, keepdims=True) - m * m
    t = (t - m) * lax.rsqrt(v + 1e-5) * ln_g[...] + ln_b[...]
    hn = jax.nn.relu(t)
    if mode == "gate":
        h = hn * aux[...]
    else:
        h = hn + aux[...]
    y = di * jnp.dot(h, w_n[...], preferred_element_type=f32)
    h_o[...] = h
    y0_o[...] = y[:, 0:16]
    y1_o[...] = y[:, 16:32]
    y2_o[...] = y[:, 32:48]
    y3_o[...] = y[:, 48:64]


def _stage_b(mode, zs, ys, dinv, aux, b_c, ln_g, ln_b, w_n):
    return pl.pallas_call(
        functools.partial(_stage_b_body, mode),
        grid=(GRID,),
        in_specs=[_rowspec(HH)] * (2 * NS)
        + [_rowspec(1), _rowspec(H), _wspec(1, H), _wspec(1, H),
           _wspec(1, H), _wspec(H, H)],
        out_specs=[_rowspec(H)] + [_rowspec(HH)] * NS,
        out_shape=[jax.ShapeDtypeStruct((N, H), f32)]
        + [jax.ShapeDtypeStruct((N, HH), f32) for _ in range(NS)],
    )(*zs, *ys, dinv, aux, b_c, ln_g, ln_b, w_n)


def _stage_c_body(z0, z1, z2, z3, y0, y1, y2, y3, dinv, hprev, res,
                  b_c, ln_g, ln_b, pre_wT, pre_b, A, post_b, out_o):
    di = dinv[...]
    t = jnp.concatenate([z0[...] + y0[...], z1[...] + y1[...],
                         z2[...] + y2[...], z3[...] + y3[...]], axis=-1)
    t = di * t + b_c[...]
    m = jnp.mean(t, -1, keepdims=True)
    v = jnp.mean(t * t, -1, keepdims=True) - m * m
    t = (t - m) * lax.rsqrt(v + 1e-5) * ln_g[...] + ln_b[...]
    h = jax.nn.relu(t) + hprev[...]
    a = jnp.tanh(jnp.dot(h, pre_wT[...], preferred_element_type=f32)
                 + pre_b[...])
    cc = jnp.cos(a * 0.5)
    ss = jnp.sin(a * 0.5)
    cols = lax.broadcasted_iota(jnp.int32, (1, 16), 1)
    psi = jnp.ones((BLK, 16), f32)
    for q in range(4):
        bit = ((cols >> (3 - q)) & 1) == 1
        psi = psi * jnp.where(bit, ss[:, q:q + 1], cc[:, q:q + 1])
    pd = jnp.sum(jnp.dot(psi, A[...], preferred_element_type=f32) * psi,
                 -1, keepdims=True) + post_b[0, 0]
    out_o[...] = pd + res[...]


def _stage_c(zs, ys, dinv, hprev, res, b_c, ln_g, ln_b,
             pre_wT, pre_b, A, post_b):
    return pl.pallas_call(
        _stage_c_body,
        grid=(GRID,),
        in_specs=[_rowspec(HH)] * (2 * NS)
        + [_rowspec(1), _rowspec(H), _rowspec(1), _wspec(1, H),
           _wspec(1, H), _wspec(1, H), _wspec(H, 4), _wspec(1, 4),
           _wspec(16, 16), _wspec(1, 1)],
        out_specs=[_rowspec(1)],
        out_shape=[jax.ShapeDtypeStruct((N, 1), f32)],
    )(*zs, *ys, dinv, hprev, res, b_c, ln_g, ln_b, pre_wT, pre_b,
      A, post_b)


def _z_diags_np():
    b = np.arange(16)
    return np.stack([1.0 - 2.0 * ((b >> (3 - i)) & 1)
                     for i in range(4)]).astype(np.float32)


def _cnot_np(c, t):
    M = np.zeros((16, 16), dtype=np.complex64)
    for b in range(16):
        b2 = b ^ (1 << (3 - t)) if (b >> (3 - c)) & 1 else b
        M[b2, b] = 1.0
    return jnp.asarray(M)


def _rot_j(phi, theta, omega):
    em = jnp.exp(-0.5j * phi).astype(jnp.complex64)
    ep = jnp.exp(0.5j * phi).astype(jnp.complex64)
    z = jnp.zeros((), jnp.complex64)
    rz1 = jnp.stack([jnp.stack([em, z]), jnp.stack([z, ep])])
    cth = jnp.cos(theta / 2).astype(jnp.complex64)
    sth = jnp.sin(theta / 2).astype(jnp.complex64)
    ry = jnp.stack([jnp.stack([cth, -sth]), jnp.stack([sth, cth])])
    em2 = jnp.exp(-0.5j * omega).astype(jnp.complex64)
    ep2 = jnp.exp(0.5j * omega).astype(jnp.complex64)
    rz2 = jnp.stack([jnp.stack([em2, z]), jnp.stack([z, ep2])])
    return rz2 @ ry @ rz1


def _quad_form(q_weights, post_w):
    U = jnp.eye(16, dtype=jnp.complex64)
    for l in range(q_weights.shape[0]):
        R = _rot_j(q_weights[l, 0, 0], q_weights[l, 0, 1], q_weights[l, 0, 2])
        for q in range(1, 4):
            R = jnp.kron(R, _rot_j(q_weights[l, q, 0], q_weights[l, q, 1],
                                   q_weights[l, q, 2]))
        U = R @ U
        r = (l % 3) + 1
        for i in range(4):
            U = _cnot_np(i, (i + r) % 4) @ U
    g = post_w[0] @ jnp.asarray(_z_diags_np())
    return jnp.real(jnp.conj(U.T) @ (g[:, None] * U))


def kernel(x, pk_embeddings, pk_predictions, edge_index, gate_w, gate_b,
           conv_w0, conv_b0, conv_w1, conv_b1, conv_w2, conv_b2,
           ln_g0, ln_b0, ln_g1, ln_b1, ln_g2, ln_b2,
           pre_w, pre_b, q_weights, post_w, post_b,
           res_w1, res_b1, res_w2, res_b2, res_alpha):
    comb = jnp.concatenate([x, pk_embeddings, pk_predictions], axis=-1)
    pad = EPAD - E
    src2d = jnp.concatenate(
        [edge_index[0], jnp.zeros((pad,), jnp.int32)]).reshape(NROWS, ROW)
    dst2d = jnp.concatenate(
        [edge_index[1], jnp.full((pad,), NPAD, jnp.int32)]).reshape(NROWS,
                                                                    ROW)

    deg0, deg1 = _deg_kernel(dst2d)

    rb2a = jnp.stack([res_b2[0], res_alpha]).reshape(1, 2)
    gate, y0, y1, y2, y3, res, dinv = _stage_a(
        comb, deg0, deg1, gate_w.T, gate_b.reshape(1, H), conv_w0,
        res_w1.T, res_b1.reshape(1, 32), res_w2.T, rb2a)

    zs = _scatter_kernel(src2d, dst2d, y0, y1, y2, y3)
    h1, y0, y1, y2, y3 = _stage_b("gate", zs, (y0, y1, y2, y3), dinv, gate,
                                  conv_b0.reshape(1, H), ln_g0.reshape(1, H),
                                  ln_b0.reshape(1, H), conv_w1)

    zs = _scatter_kernel(src2d, dst2d, y0, y1, y2, y3)
    h2, y0, y1, y2, y3 = _stage_b("res", zs, (y0, y1, y2, y3), dinv, h1,
                                  conv_b1.reshape(1, H), ln_g1.reshape(1, H),
                                  ln_b1.reshape(1, H), conv_w2)

    zs = _scatter_kernel(src2d, dst2d, y0, y1, y2, y3)
    A = _quad_form(q_weights, post_w)
    (out,) = _stage_c(zs, (y0, y1, y2, y3), dinv, h2, res,
                      conv_b2.reshape(1, H), ln_g2.reshape(1, H),
                      ln_b2.reshape(1, H), pre_w.T, pre_b.reshape(1, 4),
                      A, post_b.reshape(1, 1))
    return out


# speedup vs baseline: 11.0450x; 1.1113x over previous
import functools
import numpy as np
import jax
import jax.numpy as jnp
from jax import lax
from jax.experimental import pallas as pl
from jax.experimental.pallas import tpu as pltpu
from jax.experimental.pallas import tpu_sc as plsc

N = 50000
NPAD = 51200
ATOT = NPAD + 128
E = 800000
EPAD = 819200
H = 64
HH = 16
NS = 4
ROW = 128
NROWS = EPAD // ROW
GROUP = 8
ZCH = 128
BLK = 1000
GRID = N // BLK

_mesh = plsc.VectorSubcoreMesh(core_axis_name="c", subcore_axis_name="s")
f32 = jnp.float32
_sc_params = pltpu.CompilerParams(use_tc_tiling_on_sc=False)


@functools.partial(
    pl.kernel,
    out_type=[jax.ShapeDtypeStruct((NPAD, 16), f32),
              jax.ShapeDtypeStruct((NPAD, 16), f32)],
    mesh=_mesh,
    scratch_types=[
        pltpu.VMEM((ROW, 16), f32),
        pltpu.VMEM((ZCH, 16), f32),
        pltpu.VMEM((GROUP, ROW), jnp.int32),
        pltpu.VMEM_SHARED((ATOT, 16), f32),
    ],
    compiler_params=_sc_params,
)
def _deg_kernel(dst2d, deg0_out, deg1_out, ones_v, zero_v, idx_d, acc):
    c = lax.axis_index("c")
    s = lax.axis_index("s")
    npt = NPAD // 16

    def fill(i, _):
        ones_v[i, :] = jnp.ones((16,), f32)
        return 0
    lax.fori_loop(0, ROW, fill, 0)

    def fillz(i, _):
        zero_v[i, :] = jnp.zeros((16,), f32)
        return 0
    lax.fori_loop(0, ZCH, fillz, 0)

    def zero(i, _):
        pltpu.sync_copy(zero_v, acc.at[pl.ds(s * npt + i * ZCH, ZCH)])
        return 0
    lax.fori_loop(0, npt // ZCH, zero, 0)
    plsc.subcore_barrier()

    rows_per_tile = NROWS // 32
    base = (c * 16 + s) * rows_per_tile

    def grp(g, _):
        pltpu.sync_copy(dst2d.at[pl.ds(base + g * GROUP, GROUP)], idx_d)
        for j in range(GROUP):
            pltpu.sync_copy(ones_v, acc.at[idx_d.at[j]], add=True)
        return 0
    lax.fori_loop(0, rows_per_tile // GROUP, grp, 0)
    plsc.subcore_barrier()

    @pl.when(c == 0)
    def _():
        pltpu.sync_copy(acc.at[pl.ds(s * npt, npt)],
                        deg0_out.at[pl.ds(s * npt, npt)])

    @pl.when(c == 1)
    def _():
        pltpu.sync_copy(acc.at[pl.ds(s * npt, npt)],
                        deg1_out.at[pl.ds(s * npt, npt)])


@functools.partial(
    pl.kernel,
    out_type=[jax.ShapeDtypeStruct((NPAD, HH), f32) for _ in range(NS)],
    mesh=_mesh,
    scratch_types=[
        pltpu.VMEM((GROUP, ROW), jnp.int32),
        pltpu.VMEM((2, GROUP, ROW), jnp.int32),
        pltpu.VMEM((2, GROUP * ROW, HH), f32),
        pltpu.VMEM((ZCH, HH), f32),
        pltpu.VMEM_SHARED((ATOT, HH), f32),
        pltpu.SemaphoreType.DMA,
        pltpu.SemaphoreType.DMA,
    ],
    compiler_params=_sc_params,
)
def _scatter_kernel(src2d, dst2d, y0, y1, y2, y3, z0, z1, z2, z3,
                    idx_s, idx_d, rows, zero_v, acc, sem_g, sem_s):
    c = lax.axis_index("c")
    s = lax.axis_index("s")
    npt = NPAD // 16

    def fillz(i, _):
        zero_v[i, :] = jnp.zeros((16,), f32)
        return 0
    lax.fori_loop(0, ZCH, fillz, 0)

    rows_per_tile = NROWS // 16
    base = s * rows_per_tile

    def phase(y_ref, z_ref):
        def zero(i, _):
            pltpu.sync_copy(zero_v, acc.at[pl.ds(s * npt + i * ZCH, ZCH)])
            return 0
        lax.fori_loop(0, npt // ZCH, zero, 0)
        plsc.subcore_barrier()

        def grp(g, _):
            p = lax.rem(g, 2)
            r0 = base + g * GROUP

            @pl.when(g >= 2)
            def _():
                pltpu.make_async_copy(y_ref.at[pl.ds(0, GROUP * ROW)],
                                      rows.at[p], sem_s).wait()

            pltpu.sync_copy(src2d.at[pl.ds(r0, GROUP)], idx_s)
            pltpu.sync_copy(dst2d.at[pl.ds(r0, GROUP)], idx_d.at[p])
            descs = [pltpu.async_copy(y_ref.at[idx_s.at[j]],
                                      rows.at[p, pl.ds(j * ROW, ROW)], sem_g)
                     for j in range(GROUP)]
            for d in descs:
                d.wait()
            for j in range(GROUP):
                pltpu.async_copy(rows.at[p, pl.ds(j * ROW, ROW)],
                                 acc.at[idx_d.at[p, j]], sem_s, add=True)
            return 0
        lax.fori_loop(0, rows_per_tile // GROUP, grp, 0)
        for q in range(2):
            pltpu.make_async_copy(y_ref.at[pl.ds(0, GROUP * ROW)],
                                  rows.at[q], sem_s).wait()
        plsc.subcore_barrier()
        pltpu.sync_copy(acc.at[pl.ds(s * npt, npt)],
                        z_ref.at[pl.ds(s * npt, npt)])
        plsc.subcore_barrier()

    ys = [y0, y1, y2, y3]
    zs = [z0, z1, z2, z3]
    for q in range(2):
        @pl.when(c == 0)
        def _(q=q):
            phase(ys[q], zs[q])

        @pl.when(c == 1)
        def _(q=q):
            phase(ys[2 + q], zs[2 + q])


def _stage_a_body(comb, deg0, deg1, gate_wT, gate_b, w0, rw1T, rb1, rw2T,
                  rb2a, gate_o, y0_o, y1_o, y2_o, y3_o, res_o, dinv_o):
    x = comb[...]
    deg = deg0[:, 0:1] + deg1[:, 0:1] + 1.0
    dinv = lax.rsqrt(deg)
    g = jax.nn.sigmoid(jnp.dot(x, gate_wT[...],
                               preferred_element_type=f32) + gate_b[...])
    y = dinv * jnp.dot(x, w0[...], preferred_element_type=f32)
    r = jax.nn.relu(jnp.dot(x, rw1T[...], preferred_element_type=f32)
                    + rb1[...])
    r = jnp.dot(r, rw2T[...], preferred_element_type=f32) + rb2a[0, 0:1]
    gate_o[...] = g
    y0_o[...] = y[:, 0:16]
    y1_o[...] = y[:, 16:32]
    y2_o[...] = y[:, 32:48]
    y3_o[...] = y[:, 48:64]
    res_o[...] = r * rb2a[0, 1:2]
    dinv_o[...] = dinv


def _rowspec(k):
    return pl.BlockSpec((BLK, k), lambda i: (i, 0))


def _wspec(r, k):
    return pl.BlockSpec((r, k), lambda i: (0, 0))


def _stage_a(comb, deg0, deg1, gate_wT, gate_b, w0, rw1T, rb1, rw2T, rb2a):
    return pl.pallas_call(
        _stage_a_body,
        grid=(GRID,),
        in_specs=[_rowspec(H), _rowspec(16), _rowspec(16), _wspec(H, H),
                  _wspec(1, H), _wspec(H, H), _wspec(H, 32), _wspec(1, 32),
                  _wspec(32, 1), _wspec(1, 2)],
        out_specs=[_rowspec(H)] + [_rowspec(HH)] * NS
        + [_rowspec(1), _rowspec(1)],
        out_shape=[jax.ShapeDtypeStruct((N, H), f32)]
        + [jax.ShapeDtypeStruct((N, HH), f32) for _ in range(NS)]
        + [jax.ShapeDtypeStruct((N, 1), f32),
           jax.ShapeDtypeStruct((N, 1), f32)],
    )(comb, deg0, deg1, gate_wT, gate_b, w0, rw1T, rb1, rw2T, rb2a)


def _stage_b_body(mode, z0, z1, z2, z3, y0, y1, y2, y3, dinv, aux,
                  b_c, ln_g, ln_b, w_n, h_o, y0_o, y1_o, y2_o, y3_o):
    di = dinv[...]
    t = jnp.concatenate([z0[...] + y0[...], z1[...] + y1[...],
                         z2[...] + y2[...], z3[...] + y3[...]], axis=-1)
    t = di * t + b_c[...]
    m = jnp.mean(t, -1, keepdims=True)
    v = jnp.mean(t * t, -1, keepdims=True) - m * m
    t = (t - m) * lax.rsqrt(v + 1e-5) * ln_g[...] + ln_b[...]
    hn = jax.nn.relu(t)
    if mode == "gate":
        h = hn * aux[...]
    else:
        h = hn + aux[...]
    y = di * jnp.dot(h, w_n[...], preferred_element_type=f32)
    h_o[...] = h
    y0_o[...] = y[:, 0:16]
    y1_o[...] = y[:, 16:32]
    y2_o[...] = y[:, 32:48]
    y3_o[...] = y[:, 48:64]


def _stage_b(mode, zs, ys, dinv, aux, b_c, ln_g, ln_b, w_n):
    return pl.pallas_call(
        functools.partial(_stage_b_body, mode),
        grid=(GRID,),
        in_specs=[_rowspec(HH)] * (2 * NS)
        + [_rowspec(1), _rowspec(H), _wspec(1, H), _wspec(1, H),
           _wspec(1, H), _wspec(H, H)],
        out_specs=[_rowspec(H)] + [_rowspec(HH)] * NS,
        out_shape=[jax.ShapeDtypeStruct((N, H), f32)]
        + [jax.ShapeDtypeStruct((N, HH), f32) for _ in range(NS)],
    )(*zs, *ys, dinv, aux, b_c, ln_g, ln_b, w_n)


def _stage_c_body(z0, z1, z2, z3, y0, y1, y2, y3, dinv, hprev, res,
                  b_c, ln_g, ln_b, pre_wT, pre_b, A, post_b, out_o):
    di = dinv[...]
    t = jnp.concatenate([z0[...] + y0[...], z1[...] + y1[...],
                         z2[...] + y2[...], z3[...] + y3[...]], axis=-1)
    t = di * t + b_c[...]
    m = jnp.mean(t, -1, keepdims=True)
    v = jnp.mean(t * t, -1, keepdims=True) - m * m
    t = (t - m) * lax.rsqrt(v + 1e-5) * ln_g[...] + ln_b[...]
    h = jax.nn.relu(t) + hprev[...]
    a = jnp.tanh(jnp.dot(h, pre_wT[...], preferred_element_type=f32)
                 + pre_b[...])
    cc = jnp.cos(a * 0.5)
    ss = jnp.sin(a * 0.5)
    cols = lax.broadcasted_iota(jnp.int32, (1, 16), 1)
    psi = jnp.ones((BLK, 16), f32)
    for q in range(4):
        bit = ((cols >> (3 - q)) & 1) == 1
        psi = psi * jnp.where(bit, ss[:, q:q + 1], cc[:, q:q + 1])
    pd = jnp.sum(jnp.dot(psi, A[...], preferred_element_type=f32) * psi,
                 -1, keepdims=True) + post_b[0, 0]
    out_o[...] = pd + res[...]


def _stage_c(zs, ys, dinv, hprev, res, b_c, ln_g, ln_b,
             pre_wT, pre_b, A, post_b):
    return pl.pallas_call(
        _stage_c_body,
        grid=(GRID,),
        in_specs=[_rowspec(HH)] * (2 * NS)
        + [_rowspec(1), _rowspec(H), _rowspec(1), _wspec(1, H),
           _wspec(1, H), _wspec(1, H), _wspec(H, 4), _wspec(1, 4),
           _wspec(16, 16), _wspec(1, 1)],
        out_specs=[_rowspec(1)],
        out_shape=[jax.ShapeDtypeStruct((N, 1), f32)],
    )(*zs, *ys, dinv, hprev, res, b_c, ln_g, ln_b, pre_wT, pre_b,
      A, post_b)


def _z_diags_np():
    b = np.arange(16)
    return np.stack([1.0 - 2.0 * ((b >> (3 - i)) & 1)
                     for i in range(4)]).astype(np.float32)


def _cnot_np(c, t):
    M = np.zeros((16, 16), dtype=np.complex64)
    for b in range(16):
        b2 = b ^ (1 << (3 - t)) if (b >> (3 - c)) & 1 else b
        M[b2, b] = 1.0
    return jnp.asarray(M)


def _rot_j(phi, theta, omega):
    em = jnp.exp(-0.5j * phi).astype(jnp.complex64)
    ep = jnp.exp(0.5j * phi).astype(jnp.complex64)
    z = jnp.zeros((), jnp.complex64)
    rz1 = jnp.stack([jnp.stack([em, z]), jnp.stack([z, ep])])
    cth = jnp.cos(theta / 2).astype(jnp.complex64)
    sth = jnp.sin(theta / 2).astype(jnp.complex64)
    ry = jnp.stack([jnp.stack([cth, -sth]), jnp.stack([sth, cth])])
    em2 = jnp.exp(-0.5j * omega).astype(jnp.complex64)
    ep2 = jnp.exp(0.5j * omega).astype(jnp.complex64)
    rz2 = jnp.stack([jnp.stack([em2, z]), jnp.stack([z, ep2])])
    return rz2 @ ry @ rz1


def _quad_form(q_weights, post_w):
    U = jnp.eye(16, dtype=jnp.complex64)
    for l in range(q_weights.shape[0]):
        R = _rot_j(q_weights[l, 0, 0], q_weights[l, 0, 1], q_weights[l, 0, 2])
        for q in range(1, 4):
            R = jnp.kron(R, _rot_j(q_weights[l, q, 0], q_weights[l, q, 1],
                                   q_weights[l, q, 2]))
        U = R @ U
        r = (l % 3) + 1
        for i in range(4):
            U = _cnot_np(i, (i + r) % 4) @ U
    g = post_w[0] @ jnp.asarray(_z_diags_np())
    return jnp.real(jnp.conj(U.T) @ (g[:, None] * U))


def kernel(x, pk_embeddings, pk_predictions, edge_index, gate_w, gate_b,
           conv_w0, conv_b0, conv_w1, conv_b1, conv_w2, conv_b2,
           ln_g0, ln_b0, ln_g1, ln_b1, ln_g2, ln_b2,
           pre_w, pre_b, q_weights, post_w, post_b,
           res_w1, res_b1, res_w2, res_b2, res_alpha):
    comb = jnp.concatenate([x, pk_embeddings, pk_predictions], axis=-1)
    pad = EPAD - E
    src2d = jnp.concatenate(
        [edge_index[0], jnp.zeros((pad,), jnp.int32)]).reshape(NROWS, ROW)
    dst2d = jnp.concatenate(
        [edge_index[1], jnp.full((pad,), NPAD, jnp.int32)]).reshape(NROWS,
                                                                    ROW)

    deg0, deg1 = _deg_kernel(dst2d)

    rb2a = jnp.stack([res_b2[0], res_alpha]).reshape(1, 2)
    gate, y0, y1, y2, y3, res, dinv = _stage_a(
        comb, deg0, deg1, gate_w.T, gate_b.reshape(1, H), conv_w0,
        res_w1.T, res_b1.reshape(1, 32), res_w2.T, rb2a)

    zs = _scatter_kernel(src2d, dst2d, y0, y1, y2, y3)
    h1, y0, y1, y2, y3 = _stage_b("gate", zs, (y0, y1, y2, y3), dinv, gate,
                                  conv_b0.reshape(1, H), ln_g0.reshape(1, H),
                                  ln_b0.reshape(1, H), conv_w1)

    zs = _scatter_kernel(src2d, dst2d, y0, y1, y2, y3)
    h2, y0, y1, y2, y3 = _stage_b("res", zs, (y0, y1, y2, y3), dinv, h1,
                                  conv_b1.reshape(1, H), ln_g1.reshape(1, H),
                                  ln_b1.reshape(1, H), conv_w2)

    zs = _scatter_kernel(src2d, dst2d, y0, y1, y2, y3)
    A = _quad_form(q_weights, post_w)
    (out,) = _stage_c(zs, (y0, y1, y2, y3), dinv, h2, res,
                      conv_b2.reshape(1, H), ln_g2.reshape(1, H),
                      ln_b2.reshape(1, H), pre_w.T, pre_b.reshape(1, 4),
                      A, post_b.reshape(1, 1))
    return out


# speedup vs baseline: 12.5098x; 1.1326x over previous
import functools
import numpy as np
import jax
import jax.numpy as jnp
from jax import lax
from jax.experimental import pallas as pl
from jax.experimental.pallas import tpu as pltpu
from jax.experimental.pallas import tpu_sc as plsc

N = 50000
NPAD = 51200
ATOT = NPAD + 128
E = 800000
EPAD = 819200
H = 64
HH = 16
NS = 4
ROW = 128
NROWS = EPAD // ROW
GROUP = 8
ZCH = 128
BLK = 1000
GRID = N // BLK

_mesh = plsc.VectorSubcoreMesh(core_axis_name="c", subcore_axis_name="s")
f32 = jnp.float32
_sc_params = pltpu.CompilerParams(use_tc_tiling_on_sc=False)


@functools.partial(
    pl.kernel,
    out_type=[jax.ShapeDtypeStruct((NPAD, 16), f32),
              jax.ShapeDtypeStruct((NPAD, 16), f32)],
    mesh=_mesh,
    scratch_types=[
        pltpu.VMEM((ROW, 16), f32),
        pltpu.VMEM((ZCH, 16), f32),
        pltpu.VMEM((GROUP, ROW), jnp.int32),
        pltpu.VMEM_SHARED((ATOT, 16), f32),
    ],
    compiler_params=_sc_params,
)
def _deg_kernel(dst2d, deg0_out, deg1_out, ones_v, zero_v, idx_d, acc):
    c = lax.axis_index("c")
    s = lax.axis_index("s")
    npt = NPAD // 16

    def fill(i, _):
        ones_v[i, :] = jnp.ones((16,), f32)
        return 0
    lax.fori_loop(0, ROW, fill, 0)

    def fillz(i, _):
        zero_v[i, :] = jnp.zeros((16,), f32)
        return 0
    lax.fori_loop(0, ZCH, fillz, 0)

    def zero(i, _):
        pltpu.sync_copy(zero_v, acc.at[pl.ds(s * npt + i * ZCH, ZCH)])
        return 0
    lax.fori_loop(0, npt // ZCH, zero, 0)
    plsc.subcore_barrier()

    rows_per_tile = NROWS // 32
    base = (c * 16 + s) * rows_per_tile

    def grp(g, _):
        pltpu.sync_copy(dst2d.at[pl.ds(base + g * GROUP, GROUP)], idx_d)
        for j in range(GROUP):
            pltpu.sync_copy(ones_v, acc.at[idx_d.at[j]], add=True)
        return 0
    lax.fori_loop(0, rows_per_tile // GROUP, grp, 0)
    plsc.subcore_barrier()

    @pl.when(c == 0)
    def _():
        pltpu.sync_copy(acc.at[pl.ds(s * npt, npt)],
                        deg0_out.at[pl.ds(s * npt, npt)])

    @pl.when(c == 1)
    def _():
        pltpu.sync_copy(acc.at[pl.ds(s * npt, npt)],
                        deg1_out.at[pl.ds(s * npt, npt)])


@functools.partial(
    pl.kernel,
    out_type=[jax.ShapeDtypeStruct((NPAD, HH), f32) for _ in range(NS)],
    mesh=_mesh,
    scratch_types=[
        pltpu.VMEM((3, GROUP, ROW), jnp.int32),
        pltpu.VMEM((3, GROUP, ROW), jnp.int32),
        pltpu.VMEM((3, GROUP * ROW, HH), f32),
        pltpu.VMEM((ZCH, HH), f32),
        pltpu.VMEM_SHARED((ATOT, HH), f32),
        pltpu.SemaphoreType.DMA,
        pltpu.SemaphoreType.DMA,
    ],
    compiler_params=_sc_params,
)
def _scatter_kernel(src2d, dst2d, y0, y1, y2, y3, z0, z1, z2, z3,
                    idx_s, idx_d, rows, zero_v, acc, sem_g, sem_s):
    c = lax.axis_index("c")
    s = lax.axis_index("s")
    npt = NPAD // 16

    def fillz(i, _):
        zero_v[i, :] = jnp.zeros((16,), f32)
        return 0
    lax.fori_loop(0, ZCH, fillz, 0)

    rows_per_tile = NROWS // 16
    ngrp = rows_per_tile // GROUP
    base = s * rows_per_tile

    def phase(y_ref, z_ref):
        def zero(i, _):
            pltpu.sync_copy(zero_v, acc.at[pl.ds(s * npt + i * ZCH, ZCH)])
            return 0
        lax.fori_loop(0, npt // ZCH, zero, 0)
        plsc.subcore_barrier()

        def load_and_fire(g, b):
            r0 = base + g * GROUP
            pltpu.sync_copy(src2d.at[pl.ds(r0, GROUP)], idx_s.at[b])
            pltpu.sync_copy(dst2d.at[pl.ds(r0, GROUP)], idx_d.at[b])
            for j in range(GROUP):
                pltpu.async_copy(y_ref.at[idx_s.at[b, j]],
                                 rows.at[b, pl.ds(j * ROW, ROW)], sem_g)

        load_and_fire(0, 0)

        def grp(g, _):
            p = lax.rem(g, 3)
            pn = lax.rem(g + 1, 3)

            @pl.when(g + 1 < ngrp)
            def _():
                @pl.when(g >= 2)
                def _():
                    pltpu.make_async_copy(y_ref.at[pl.ds(0, GROUP * ROW)],
                                          rows.at[pn], sem_s).wait()
                load_and_fire(g + 1, pn)

            pltpu.make_async_copy(y_ref.at[pl.ds(0, GROUP * ROW)],
                                  rows.at[p], sem_g).wait()
            for j in range(GROUP):
                pltpu.async_copy(rows.at[p, pl.ds(j * ROW, ROW)],
                                 acc.at[idx_d.at[p, j]], sem_s, add=True)
            return 0
        lax.fori_loop(0, ngrp, grp, 0)
        for q in range(3):
            pltpu.make_async_copy(y_ref.at[pl.ds(0, GROUP * ROW)],
                                  rows.at[q], sem_s).wait()
        plsc.subcore_barrier()
        pltpu.sync_copy(acc.at[pl.ds(s * npt, npt)],
                        z_ref.at[pl.ds(s * npt, npt)])
        plsc.subcore_barrier()

    ys = [y0, y1, y2, y3]
    zs = [z0, z1, z2, z3]
    for q in range(2):
        @pl.when(c == 0)
        def _(q=q):
            phase(ys[q], zs[q])

        @pl.when(c == 1)
        def _(q=q):
            phase(ys[2 + q], zs[2 + q])


def _stage_a_body(comb, deg0, deg1, gate_wT, gate_b, w0, rw1T, rb1, rw2T,
                  rb2a, gate_o, y0_o, y1_o, y2_o, y3_o, res_o, dinv_o):
    x = comb[...]
    deg = deg0[:, 0:1] + deg1[:, 0:1] + 1.0
    dinv = lax.rsqrt(deg)
    g = jax.nn.sigmoid(jnp.dot(x, gate_wT[...],
                               preferred_element_type=f32) + gate_b[...])
    y = dinv * jnp.dot(x, w0[...], preferred_element_type=f32)
    r = jax.nn.relu(jnp.dot(x, rw1T[...], preferred_element_type=f32)
                    + rb1[...])
    r = jnp.dot(r, rw2T[...], preferred_element_type=f32) + rb2a[0, 0:1]
    gate_o[...] = g
    y0_o[...] = y[:, 0:16]
    y1_o[...] = y[:, 16:32]
    y2_o[...] = y[:, 32:48]
    y3_o[...] = y[:, 48:64]
    res_o[...] = r * rb2a[0, 1:2]
    dinv_o[...] = dinv


def _rowspec(k):
    return pl.BlockSpec((BLK, k), lambda i: (i, 0))


def _wspec(r, k):
    return pl.BlockSpec((r, k), lambda i: (0, 0))


def _stage_a(comb, deg0, deg1, gate_wT, gate_b, w0, rw1T, rb1, rw2T, rb2a):
    return pl.pallas_call(
        _stage_a_body,
        grid=(GRID,),
        in_specs=[_rowspec(H), _rowspec(16), _rowspec(16), _wspec(H, H),
                  _wspec(1, H), _wspec(H, H), _wspec(H, 32), _wspec(1, 32),
                  _wspec(32, 1), _wspec(1, 2)],
        out_specs=[_rowspec(H)] + [_rowspec(HH)] * NS
        + [_rowspec(1), _rowspec(1)],
        out_shape=[jax.ShapeDtypeStruct((N, H), f32)]
        + [jax.ShapeDtypeStruct((N, HH), f32) for _ in range(NS)]
        + [jax.ShapeDtypeStruct((N, 1), f32),
           jax.ShapeDtypeStruct((N, 1), f32)],
    )(comb, deg0, deg1, gate_wT, gate_b, w0, rw1T, rb1, rw2T, rb2a)


def _stage_b_body(mode, z0, z1, z2, z3, y0, y1, y2, y3, dinv, aux,
                  b_c, ln_g, ln_b, w_n, h_o, y0_o, y1_o, y2_o, y3_o):
    di = dinv[...]
    t = jnp.concatenate([z0[...] + y0[...], z1[...] + y1[...],
                         z2[...] + y2[...], z3[...] + y3[...]], axis=-1)
    t = di * t + b_c[...]
    m = jnp.mean(t, -1, keepdims=True)
    v = jnp.mean(t * t, -1, keepdims=True) - m * m
    t = (t - m) * lax.rsqrt(v + 1e-5) * ln_g[...] + ln_b[...]
    hn = jax.nn.relu(t)
    if mode == "gate":
        h = hn * aux[...]
    else:
        h = hn + aux[...]
    y = di * jnp.dot(h, w_n[...], preferred_element_type=f32)
    h_o[...] = h
    y0_o[...] = y[:, 0:16]
    y1_o[...] = y[:, 16:32]
    y2_o[...] = y[:, 32:48]
    y3_o[...] = y[:, 48:64]


def _stage_b(mode, zs, ys, dinv, aux, b_c, ln_g, ln_b, w_n):
    return pl.pallas_call(
        functools.partial(_stage_b_body, mode),
        grid=(GRID,),
        in_specs=[_rowspec(HH)] * (2 * NS)
        + [_rowspec(1), _rowspec(H), _wspec(1, H), _wspec(1, H),
           _wspec(1, H), _wspec(H, H)],
        out_specs=[_rowspec(H)] + [_rowspec(HH)] * NS,
        out_shape=[jax.ShapeDtypeStruct((N, H), f32)]
        + [jax.ShapeDtypeStruct((N, HH), f32) for _ in range(NS)],
    )(*zs, *ys, dinv, aux, b_c, ln_g, ln_b, w_n)


def _stage_c_body(z0, z1, z2, z3, y0, y1, y2, y3, dinv, hprev, res,
                  b_c, ln_g, ln_b, pre_wT, pre_b, A, post_b, out_o):
    di = dinv[...]
    t = jnp.concatenate([z0[...] + y0[...], z1[...] + y1[...],
                         z2[...] + y2[...], z3[...] + y3[...]], axis=-1)
    t = di * t + b_c[...]
    m = jnp.mean(t, -1, keepdims=True)
    v = jnp.mean(t * t, -1, keepdims=True) - m * m
    t = (t - m) * lax.rsqrt(v + 1e-5) * ln_g[...] + ln_b[...]
    h = jax.nn.relu(t) + hprev[...]
    a = jnp.tanh(jnp.dot(h, pre_wT[...], preferred_element_type=f32)
                 + pre_b[...])
    cc = jnp.cos(a * 0.5)
    ss = jnp.sin(a * 0.5)
    cols = lax.broadcasted_iota(jnp.int32, (1, 16), 1)
    psi = jnp.ones((BLK, 16), f32)
    for q in range(4):
        bit = ((cols >> (3 - q)) & 1) == 1
        psi = psi * jnp.where(bit, ss[:, q:q + 1], cc[:, q:q + 1])
    pd = jnp.sum(jnp.dot(psi, A[...], preferred_element_type=f32) * psi,
                 -1, keepdims=True) + post_b[0, 0]
    out_o[...] = pd + res[...]


def _stage_c(zs, ys, dinv, hprev, res, b_c, ln_g, ln_b,
             pre_wT, pre_b, A, post_b):
    return pl.pallas_call(
        _stage_c_body,
        grid=(GRID,),
        in_specs=[_rowspec(HH)] * (2 * NS)
        + [_rowspec(1), _rowspec(H), _rowspec(1), _wspec(1, H),
           _wspec(1, H), _wspec(1, H), _wspec(H, 4), _wspec(1, 4),
           _wspec(16, 16), _wspec(1, 1)],
        out_specs=[_rowspec(1)],
        out_shape=[jax.ShapeDtypeStruct((N, 1), f32)],
    )(*zs, *ys, dinv, hprev, res, b_c, ln_g, ln_b, pre_wT, pre_b,
      A, post_b)


def _z_diags_np():
    b = np.arange(16)
    return np.stack([1.0 - 2.0 * ((b >> (3 - i)) & 1)
                     for i in range(4)]).astype(np.float32)


def _cnot_np(c, t):
    M = np.zeros((16, 16), dtype=np.complex64)
    for b in range(16):
        b2 = b ^ (1 << (3 - t)) if (b >> (3 - c)) & 1 else b
        M[b2, b] = 1.0
    return jnp.asarray(M)


def _rot_j(phi, theta, omega):
    em = jnp.exp(-0.5j * phi).astype(jnp.complex64)
    ep = jnp.exp(0.5j * phi).astype(jnp.complex64)
    z = jnp.zeros((), jnp.complex64)
    rz1 = jnp.stack([jnp.stack([em, z]), jnp.stack([z, ep])])
    cth = jnp.cos(theta / 2).astype(jnp.complex64)
    sth = jnp.sin(theta / 2).astype(jnp.complex64)
    ry = jnp.stack([jnp.stack([cth, -sth]), jnp.stack([sth, cth])])
    em2 = jnp.exp(-0.5j * omega).astype(jnp.complex64)
    ep2 = jnp.exp(0.5j * omega).astype(jnp.complex64)
    rz2 = jnp.stack([jnp.stack([em2, z]), jnp.stack([z, ep2])])
    return rz2 @ ry @ rz1


def _quad_form(q_weights, post_w):
    U = jnp.eye(16, dtype=jnp.complex64)
    for l in range(q_weights.shape[0]):
        R = _rot_j(q_weights[l, 0, 0], q_weights[l, 0, 1], q_weights[l, 0, 2])
        for q in range(1, 4):
            R = jnp.kron(R, _rot_j(q_weights[l, q, 0], q_weights[l, q, 1],
                                   q_weights[l, q, 2]))
        U = R @ U
        r = (l % 3) + 1
        for i in range(4):
            U = _cnot_np(i, (i + r) % 4) @ U
    g = post_w[0] @ jnp.asarray(_z_diags_np())
    return jnp.real(jnp.conj(U.T) @ (g[:, None] * U))


def kernel(x, pk_embeddings, pk_predictions, edge_index, gate_w, gate_b,
           conv_w0, conv_b0, conv_w1, conv_b1, conv_w2, conv_b2,
           ln_g0, ln_b0, ln_g1, ln_b1, ln_g2, ln_b2,
           pre_w, pre_b, q_weights, post_w, post_b,
           res_w1, res_b1, res_w2, res_b2, res_alpha):
    comb = jnp.concatenate([x, pk_embeddings, pk_predictions], axis=-1)
    pad = EPAD - E
    src2d = jnp.concatenate(
        [edge_index[0], jnp.zeros((pad,), jnp.int32)]).reshape(NROWS, ROW)
    dst2d = jnp.concatenate(
        [edge_index[1], jnp.full((pad,), NPAD, jnp.int32)]).reshape(NROWS,
                                                                    ROW)

    deg0, deg1 = _deg_kernel(dst2d)

    rb2a = jnp.stack([res_b2[0], res_alpha]).reshape(1, 2)
    gate, y0, y1, y2, y3, res, dinv = _stage_a(
        comb, deg0, deg1, gate_w.T, gate_b.reshape(1, H), conv_w0,
        res_w1.T, res_b1.reshape(1, 32), res_w2.T, rb2a)

    zs = _scatter_kernel(src2d, dst2d, y0, y1, y2, y3)
    h1, y0, y1, y2, y3 = _stage_b("gate", zs, (y0, y1, y2, y3), dinv, gate,
                                  conv_b0.reshape(1, H), ln_g0.reshape(1, H),
                                  ln_b0.reshape(1, H), conv_w1)

    zs = _scatter_kernel(src2d, dst2d, y0, y1, y2, y3)
    h2, y0, y1, y2, y3 = _stage_b("res", zs, (y0, y1, y2, y3), dinv, h1,
                                  conv_b1.reshape(1, H), ln_g1.reshape(1, H),
                                  ln_b1.reshape(1, H), conv_w2)

    zs = _scatter_kernel(src2d, dst2d, y0, y1, y2, y3)
    A = _quad_form(q_weights, post_w)
    (out,) = _stage_c(zs, (y0, y1, y2, y3), dinv, h2, res,
                      conv_b2.reshape(1, H), ln_g2.reshape(1, H),
                      ln_b2.reshape(1, H), pre_w.T, pre_b.reshape(1, 4),
                      A, post_b.reshape(1, 1))
    return out


# speedup vs baseline: 12.5116x; 1.0001x over previous
import functools
import numpy as np
import jax
import jax.numpy as jnp
from jax import lax
from jax.experimental import pallas as pl
from jax.experimental.pallas import tpu as pltpu
from jax.experimental.pallas import tpu_sc as plsc

N = 50000
NPAD = 51200
ATOT = NPAD + 128
E = 800000
EPAD = 819200
H = 64
HH = 16
NS = 4
ROW = 128
NROWS = EPAD // ROW
GROUP = 8
ZCH = 128
BLK = 1000
GRID = N // BLK

_mesh = plsc.VectorSubcoreMesh(core_axis_name="c", subcore_axis_name="s")
f32 = jnp.float32
_sc_params = pltpu.CompilerParams(use_tc_tiling_on_sc=False)


@functools.partial(
    pl.kernel,
    out_type=[jax.ShapeDtypeStruct((NPAD, 16), f32),
              jax.ShapeDtypeStruct((NPAD, 16), f32)],
    mesh=_mesh,
    scratch_types=[
        pltpu.VMEM((ROW, 16), f32),
        pltpu.VMEM((ZCH, 16), f32),
        pltpu.VMEM((GROUP, ROW), jnp.int32),
        pltpu.VMEM_SHARED((ATOT, 16), f32),
    ],
    compiler_params=_sc_params,
)
def _deg_kernel(dst2d, deg0_out, deg1_out, ones_v, zero_v, idx_d, acc):
    c = lax.axis_index("c")
    s = lax.axis_index("s")
    npt = NPAD // 16

    def fill(i, _):
        ones_v[i, :] = jnp.ones((16,), f32)
        return 0
    lax.fori_loop(0, ROW, fill, 0)

    def fillz(i, _):
        zero_v[i, :] = jnp.zeros((16,), f32)
        return 0
    lax.fori_loop(0, ZCH, fillz, 0)

    def zero(i, _):
        pltpu.sync_copy(zero_v, acc.at[pl.ds(s * npt + i * ZCH, ZCH)])
        return 0
    lax.fori_loop(0, npt // ZCH, zero, 0)
    plsc.subcore_barrier()

    rows_per_tile = NROWS // 32
    base = (c * 16 + s) * rows_per_tile

    def grp(g, _):
        pltpu.sync_copy(dst2d.at[pl.ds(base + g * GROUP, GROUP)], idx_d)
        for j in range(GROUP):
            pltpu.sync_copy(ones_v, acc.at[idx_d.at[j]], add=True)
        return 0
    lax.fori_loop(0, rows_per_tile // GROUP, grp, 0)
    plsc.subcore_barrier()

    @pl.when(c == 0)
    def _():
        pltpu.sync_copy(acc.at[pl.ds(s * npt, npt)],
                        deg0_out.at[pl.ds(s * npt, npt)])

    @pl.when(c == 1)
    def _():
        pltpu.sync_copy(acc.at[pl.ds(s * npt, npt)],
                        deg1_out.at[pl.ds(s * npt, npt)])


@functools.partial(
    pl.kernel,
    out_type=jax.ShapeDtypeStruct((4 * NPAD, HH), f32),
    mesh=_mesh,
    scratch_types=[
        pltpu.VMEM((3, GROUP, ROW), jnp.int32),
        pltpu.VMEM((3, GROUP, ROW), jnp.int32),
        pltpu.VMEM((3, GROUP * ROW, HH), f32),
        pltpu.VMEM((2, ROW), jnp.int32),
        pltpu.VMEM((ZCH, HH), f32),
        pltpu.VMEM_SHARED((ATOT, HH), f32),
        pltpu.SemaphoreType.DMA,
        pltpu.SemaphoreType.DMA,
    ],
    compiler_params=_sc_params,
)
def _scatter_kernel(src4, dst2d, idxz, y4, z4,
                    idx_s, idx_d, rows, idxz_v, zero_v, acc, sem_g, sem_s):
    c = lax.axis_index("c")
    s = lax.axis_index("s")
    npt = NPAD // 16

    def fillz(i, _):
        zero_v[i, :] = jnp.zeros((16,), f32)
        return 0
    lax.fori_loop(0, ZCH, fillz, 0)

    rows_per_tile = NROWS // 16
    ngrp = rows_per_tile // GROUP
    base = s * rows_per_tile

    def phase(q):
        def zero(i, _):
            pltpu.sync_copy(zero_v, acc.at[pl.ds(s * npt + i * ZCH, ZCH)])
            return 0
        lax.fori_loop(0, npt // ZCH, zero, 0)
        plsc.subcore_barrier()

        def load_and_fire(g, b):
            r0 = base + g * GROUP
            pltpu.sync_copy(src4.at[q, pl.ds(r0, GROUP)], idx_s.at[b])
            pltpu.sync_copy(dst2d.at[pl.ds(r0, GROUP)], idx_d.at[b])
            for j in range(GROUP):
                pltpu.async_copy(y4.at[idx_s.at[b, j]],
                                 rows.at[b, pl.ds(j * ROW, ROW)], sem_g)

        load_and_fire(0, 0)

        def grp(g, _):
            p = lax.rem(g, 3)
            pn = lax.rem(g + 1, 3)

            @pl.when(g + 1 < ngrp)
            def _():
                @pl.when(g >= 2)
                def _():
                    pltpu.make_async_copy(y4.at[pl.ds(0, GROUP * ROW)],
                                          rows.at[pn], sem_s).wait()
                load_and_fire(g + 1, pn)

            pltpu.make_async_copy(y4.at[pl.ds(0, GROUP * ROW)],
                                  rows.at[p], sem_g).wait()
            for j in range(GROUP):
                pltpu.async_copy(rows.at[p, pl.ds(j * ROW, ROW)],
                                 acc.at[idx_d.at[p, j]], sem_s, add=True)
            return 0
        lax.fori_loop(0, ngrp, grp, 0)
        for r in range(3):
            pltpu.make_async_copy(y4.at[pl.ds(0, GROUP * ROW)],
                                  rows.at[r], sem_s).wait()
        plsc.subcore_barrier()
        nch = npt // ROW
        def cout(i, _):
            b = lax.rem(i, 2)

            @pl.when(i >= 2)
            def _():
                pltpu.make_async_copy(y4.at[pl.ds(0, ROW)],
                                      rows.at[2, pl.ds(0, ROW)], sem_s).wait()
            pltpu.sync_copy(idxz.at[q, s * nch + i], idxz_v.at[b])
            pltpu.sync_copy(acc.at[pl.ds(s * npt + i * ROW, ROW)],
                            rows.at[b, pl.ds(0, ROW)])
            pltpu.async_copy(rows.at[b, pl.ds(0, ROW)],
                             z4.at[idxz_v.at[b]], sem_s)
            return 0
        lax.fori_loop(0, nch, cout, 0)
        for r in range(2):
            pltpu.make_async_copy(y4.at[pl.ds(0, ROW)],
                                  rows.at[0, pl.ds(0, ROW)], sem_s).wait()
        plsc.subcore_barrier()

    for qq in range(2):
        @pl.when(c == 0)
        def _(qq=qq):
            phase(qq)

        @pl.when(c == 1)
        def _(qq=qq):
            phase(2 + qq)


def _stage_a_body(comb, deg0, deg1, gate_wT, gate_b, w0, rw1T, rb1, rw2T,
                  rb2a, gate_o, y_o, ext_o):
    x = comb[...]
    deg = deg0[:, 0:1] + deg1[:, 0:1] + 1.0
    dinv = lax.rsqrt(deg)
    g = jax.nn.sigmoid(jnp.dot(x, gate_wT[...],
                               preferred_element_type=f32) + gate_b[...])
    y = dinv * jnp.dot(x, w0[...], preferred_element_type=f32)
    r = jax.nn.relu(jnp.dot(x, rw1T[...], preferred_element_type=f32)
                    + rb1[...])
    r = jnp.dot(r, rw2T[...], preferred_element_type=f32) + rb2a[0, 0:1]
    gate_o[...] = g
    y_o[...] = y
    ext_o[...] = jnp.concatenate(
        [dinv, r * rb2a[0, 1:2], jnp.zeros((BLK, H - 2), f32)], axis=-1)


def _rowspec(k):
    return pl.BlockSpec((BLK, k), lambda i: (i, 0))


def _wspec(r, k):
    return pl.BlockSpec((r, k), lambda i: (0, 0))


def _stage_a(comb, deg0, deg1, gate_wT, gate_b, w0, rw1T, rb1, rw2T, rb2a):
    return pl.pallas_call(
        _stage_a_body,
        grid=(GRID,),
        in_specs=[_rowspec(H), _rowspec(16), _rowspec(16), _wspec(H, H),
                  _wspec(1, H), _wspec(H, H), _wspec(H, 32), _wspec(1, 32),
                  _wspec(32, 1), _wspec(1, 2)],
        out_specs=[_rowspec(H), _rowspec(H), _rowspec(H)],
        out_shape=[jax.ShapeDtypeStruct((N, H), f32) for _ in range(3)],
    )(comb, deg0, deg1, gate_wT, gate_b, w0, rw1T, rb1, rw2T, rb2a)


def _stage_b_body(mode, z, y, ext, aux, b_c, ln_g, ln_b, w_n, h_o, y_o):
    di = ext[:, 0:1]
    t = di * (z[...] + y[...]) + b_c[...]
    m = jnp.mean(t, -1, keepdims=True)
    v = jnp.mean(t * t, -1, keepdims=True) - m * m
    t = (t - m) * lax.rsqrt(v + 1e-5) * ln_g[...] + ln_b[...]
    hn = jax.nn.relu(t)
    if mode == "gate":
        h = hn * aux[...]
    else:
        h = hn + aux[...]
    y_o[...] = di * jnp.dot(h, w_n[...], preferred_element_type=f32)
    h_o[...] = h


def _stage_b(mode, z, y, ext, aux, b_c, ln_g, ln_b, w_n):
    return pl.pallas_call(
        functools.partial(_stage_b_body, mode),
        grid=(GRID,),
        in_specs=[_rowspec(H), _rowspec(H), _rowspec(H), _rowspec(H),
                  _wspec(1, H), _wspec(1, H), _wspec(1, H), _wspec(H, H)],
        out_specs=[_rowspec(H), _rowspec(H)],
        out_shape=[jax.ShapeDtypeStruct((N, H), f32) for _ in range(2)],
    )(z, y, ext, aux, b_c, ln_g, ln_b, w_n)


def _stage_c_body(z, y, ext, hprev, b_c, ln_g, ln_b, pre_wT, pre_b, A,
                  post_b, out_o):
    di = ext[:, 0:1]
    t = di * (z[...] + y[...]) + b_c[...]
    m = jnp.mean(t, -1, keepdims=True)
    v = jnp.mean(t * t, -1, keepdims=True) - m * m
    t = (t - m) * lax.rsqrt(v + 1e-5) * ln_g[...] + ln_b[...]
    h = jax.nn.relu(t) + hprev[...]
    a = jnp.tanh(jnp.dot(h, pre_wT[...], preferred_element_type=f32)
                 + pre_b[...])
    cc = jnp.cos(a * 0.5)
    ss = jnp.sin(a * 0.5)
    cols = lax.broadcasted_iota(jnp.int32, (1, 16), 1)
    psi = jnp.ones((BLK, 16), f32)
    for q in range(4):
        bit = ((cols >> (3 - q)) & 1) == 1
        psi = psi * jnp.where(bit, ss[:, q:q + 1], cc[:, q:q + 1])
    pd = jnp.sum(jnp.dot(psi, A[...], preferred_element_type=f32) * psi,
                 -1, keepdims=True) + post_b[0, 0]
    out_o[...] = pd + ext[:, 1:2]


def _stage_c(z, y, ext, hprev, b_c, ln_g, ln_b, pre_wT, pre_b, A, post_b):
    return pl.pallas_call(
        _stage_c_body,
        grid=(GRID,),
        in_specs=[_rowspec(H), _rowspec(H), _rowspec(H), _rowspec(H),
                  _wspec(1, H), _wspec(1, H), _wspec(1, H), _wspec(H, 4),
                  _wspec(1, 4), _wspec(16, 16), _wspec(1, 1)],
        out_specs=[_rowspec(1)],
        out_shape=[jax.ShapeDtypeStruct((N, 1), f32)],
    )(z, y, ext, hprev, b_c, ln_g, ln_b, pre_wT, pre_b, A, post_b)


def _z_diags_np():
    b = np.arange(16)
    return np.stack([1.0 - 2.0 * ((b >> (3 - i)) & 1)
                     for i in range(4)]).astype(np.float32)


def _cnot_np(c, t):
    M = np.zeros((16, 16), dtype=np.complex64)
    for b in range(16):
        b2 = b ^ (1 << (3 - t)) if (b >> (3 - c)) & 1 else b
        M[b2, b] = 1.0
    return jnp.asarray(M)


def _rot_j(phi, theta, omega):
    em = jnp.exp(-0.5j * phi).astype(jnp.complex64)
    ep = jnp.exp(0.5j * phi).astype(jnp.complex64)
    z = jnp.zeros((), jnp.complex64)
    rz1 = jnp.stack([jnp.stack([em, z]), jnp.stack([z, ep])])
    cth = jnp.cos(theta / 2).astype(jnp.complex64)
    sth = jnp.sin(theta / 2).astype(jnp.complex64)
    ry = jnp.stack([jnp.stack([cth, -sth]), jnp.stack([sth, cth])])
    em2 = jnp.exp(-0.5j * omega).astype(jnp.complex64)
    ep2 = jnp.exp(0.5j * omega).astype(jnp.complex64)
    rz2 = jnp.stack([jnp.stack([em2, z]), jnp.stack([z, ep2])])
    return rz2 @ ry @ rz1


def _quad_form(q_weights, post_w):
    U = jnp.eye(16, dtype=jnp.complex64)
    for l in range(q_weights.shape[0]):
        R = _rot_j(q_weights[l, 0, 0], q_weights[l, 0, 1], q_weights[l, 0, 2])
        for q in range(1, 4):
            R = jnp.kron(R, _rot_j(q_weights[l, q, 0], q_weights[l, q, 1],
                                   q_weights[l, q, 2]))
        U = R @ U
        r = (l % 3) + 1
        for i in range(4):
            U = _cnot_np(i, (i + r) % 4) @ U
    g = post_w[0] @ jnp.asarray(_z_diags_np())
    return jnp.real(jnp.conj(U.T) @ (g[:, None] * U))


def kernel(x, pk_embeddings, pk_predictions, edge_index, gate_w, gate_b,
           conv_w0, conv_b0, conv_w1, conv_b1, conv_w2, conv_b2,
           ln_g0, ln_b0, ln_g1, ln_b1, ln_g2, ln_b2,
           pre_w, pre_b, q_weights, post_w, post_b,
           res_w1, res_b1, res_w2, res_b2, res_alpha):
    comb = jnp.concatenate([x, pk_embeddings, pk_predictions], axis=-1)
    pad = EPAD - E
    src2d = jnp.concatenate(
        [edge_index[0], jnp.zeros((pad,), jnp.int32)]).reshape(NROWS, ROW)
    dst2d = jnp.concatenate(
        [edge_index[1], jnp.full((pad,), NPAD, jnp.int32)]).reshape(NROWS,
                                                                    ROW)
    src4 = (4 * src2d)[None, :, :] + jnp.arange(4, dtype=jnp.int32).reshape(
        4, 1, 1)
    idxz = (4 * jnp.arange(NPAD, dtype=jnp.int32).reshape(1, NPAD // ROW,
                                                          ROW)
            + jnp.arange(4, dtype=jnp.int32).reshape(4, 1, 1))

    deg0, deg1 = _deg_kernel(dst2d)

    rb2a = jnp.stack([res_b2[0], res_alpha]).reshape(1, 2)
    gate, y64, ext = _stage_a(
        comb, deg0, deg1, gate_w.T, gate_b.reshape(1, H), conv_w0,
        res_w1.T, res_b1.reshape(1, 32), res_w2.T, rb2a)

    z64 = _scatter_kernel(src4, dst2d, idxz,
                          y64.reshape(4 * N, HH)).reshape(NPAD, H)
    h1, y64 = _stage_b("gate", z64, y64, ext, gate,
                       conv_b0.reshape(1, H), ln_g0.reshape(1, H),
                       ln_b0.reshape(1, H), conv_w1)

    z64 = _scatter_kernel(src4, dst2d, idxz,
                          y64.reshape(4 * N, HH)).reshape(NPAD, H)
    h2, y64 = _stage_b("res", z64, y64, ext, h1,
                       conv_b1.reshape(1, H), ln_g1.reshape(1, H),
                       ln_b1.reshape(1, H), conv_w2)

    z64 = _scatter_kernel(src4, dst2d, idxz,
                          y64.reshape(4 * N, HH)).reshape(NPAD, H)
    A = _quad_form(q_weights, post_w)
    (out,) = _stage_c(z64, y64, ext, h2,
                      conv_b2.reshape(1, H), ln_g2.reshape(1, H),
                      ln_b2.reshape(1, H), pre_w.T, pre_b.reshape(1, 4),
                      A, post_b.reshape(1, 1))
    return out


# speedup vs baseline: 13.0194x; 1.0406x over previous
import functools
import numpy as np
import jax
import jax.numpy as jnp
from jax import lax
from jax.experimental import pallas as pl
from jax.experimental.pallas import tpu as pltpu
from jax.experimental.pallas import tpu_sc as plsc

N = 50000
NPAD = 51200
ATOT = NPAD + 128
E = 800000
EPAD = 819200
H = 64
HH = 16
NS = 4
ROW = 128
NROWS = EPAD // ROW
GROUP = 8
ZCH = 128
BLK = 1000
GRID = N // BLK

_mesh = plsc.VectorSubcoreMesh(core_axis_name="c", subcore_axis_name="s")
f32 = jnp.float32
_sc_params = pltpu.CompilerParams(use_tc_tiling_on_sc=False)


@functools.partial(
    pl.kernel,
    out_type=[jax.ShapeDtypeStruct((NPAD, 16), f32),
              jax.ShapeDtypeStruct((NPAD, 16), f32)],
    mesh=_mesh,
    scratch_types=[
        pltpu.VMEM((ROW, 16), f32),
        pltpu.VMEM((ZCH, 16), f32),
        pltpu.VMEM((GROUP, ROW), jnp.int32),
        pltpu.VMEM_SHARED((ATOT, 16), f32),
    ],
    compiler_params=_sc_params,
)
def _deg_kernel(dst2d, deg0_out, deg1_out, ones_v, zero_v, idx_d, acc):
    c = lax.axis_index("c")
    s = lax.axis_index("s")
    npt = NPAD // 16

    def fill(i, _):
        ones_v[i, :] = jnp.ones((16,), f32)
        return 0
    lax.fori_loop(0, ROW, fill, 0)

    def fillz(i, _):
        zero_v[i, :] = jnp.zeros((16,), f32)
        return 0
    lax.fori_loop(0, ZCH, fillz, 0)

    def zero(i, _):
        pltpu.sync_copy(zero_v, acc.at[pl.ds(s * npt + i * ZCH, ZCH)])
        return 0
    lax.fori_loop(0, npt // ZCH, zero, 0)
    plsc.subcore_barrier()

    rows_per_tile = NROWS // 32
    base = (c * 16 + s) * rows_per_tile

    def grp(g, _):
        pltpu.sync_copy(dst2d.at[pl.ds(base + g * GROUP, GROUP)], idx_d)
        for j in range(GROUP):
            pltpu.sync_copy(ones_v, acc.at[idx_d.at[j]], add=True)
        return 0
    lax.fori_loop(0, rows_per_tile // GROUP, grp, 0)
    plsc.subcore_barrier()

    @pl.when(c == 0)
    def _():
        pltpu.sync_copy(acc.at[pl.ds(s * npt, npt)],
                        deg0_out.at[pl.ds(s * npt, npt)])

    @pl.when(c == 1)
    def _():
        pltpu.sync_copy(acc.at[pl.ds(s * npt, npt)],
                        deg1_out.at[pl.ds(s * npt, npt)])


@functools.partial(
    pl.kernel,
    out_type=jax.ShapeDtypeStruct((4 * NPAD, HH), f32),
    mesh=_mesh,
    scratch_types=[
        pltpu.VMEM((3, GROUP, ROW), jnp.int32),
        pltpu.VMEM((3, GROUP, ROW), jnp.int32),
        pltpu.VMEM((3, GROUP * ROW, HH), f32),
        pltpu.VMEM((32, ROW), jnp.int32),
        pltpu.VMEM((ZCH, HH), f32),
        pltpu.VMEM_SHARED((ATOT, HH), f32),
        pltpu.SemaphoreType.DMA,
        pltpu.SemaphoreType.DMA,
    ],
    compiler_params=_sc_params,
)
def _scatter_kernel(src4, dst2d, idxz, y4, z4,
                    idx_s, idx_d, rows, idxz_v, zero_v, acc, sem_g, sem_s):
    c = lax.axis_index("c")
    s = lax.axis_index("s")
    npt = NPAD // 16

    def fillz(i, _):
        zero_v[i, :] = jnp.zeros((16,), f32)
        return 0
    lax.fori_loop(0, ZCH, fillz, 0)

    rows_per_tile = NROWS // 16
    ngrp = rows_per_tile // GROUP
    base = s * rows_per_tile

    def phase(q):
        def zero(i, _):
            pltpu.sync_copy(zero_v, acc.at[pl.ds(s * npt + i * ZCH, ZCH)])
            return 0
        lax.fori_loop(0, npt // ZCH, zero, 0)
        plsc.subcore_barrier()

        def load_and_fire(g, b):
            r0 = base + g * GROUP
            pltpu.sync_copy(src4.at[q, pl.ds(r0, GROUP)], idx_s.at[b])
            pltpu.sync_copy(dst2d.at[pl.ds(r0, GROUP)], idx_d.at[b])
            for j in range(GROUP):
                pltpu.async_copy(y4.at[idx_s.at[b, j]],
                                 rows.at[b, pl.ds(j * ROW, ROW)], sem_g)

        load_and_fire(0, 0)

        def grp(g, _):
            p = lax.rem(g, 3)
            pn = lax.rem(g + 1, 3)

            @pl.when(g + 1 < ngrp)
            def _():
                @pl.when(g >= 2)
                def _():
                    pltpu.make_async_copy(y4.at[pl.ds(0, GROUP * ROW)],
                                          rows.at[pn], sem_s).wait()
                load_and_fire(g + 1, pn)

            pltpu.make_async_copy(y4.at[pl.ds(0, GROUP * ROW)],
                                  rows.at[p], sem_g).wait()
            for j in range(GROUP):
                pltpu.async_copy(rows.at[p, pl.ds(j * ROW, ROW)],
                                 acc.at[idx_d.at[p, j]], sem_s, add=True)
            return 0
        lax.fori_loop(0, ngrp, grp, 0)
        for r in range(3):
            pltpu.make_async_copy(y4.at[pl.ds(0, GROUP * ROW)],
                                  rows.at[r], sem_s).wait()
        plsc.subcore_barrier()
        CPB = 5
        pltpu.sync_copy(idxz.at[q, s], idxz_v)

        def cblk(blk, _):
            b = lax.rem(blk, 2)

            @pl.when(blk >= 2)
            def _():
                pltpu.make_async_copy(y4.at[pl.ds(0, CPB * ROW)],
                                      rows.at[2, pl.ds(0, CPB * ROW)],
                                      sem_s).wait()
            pltpu.sync_copy(acc.at[pl.ds(s * npt + blk * CPB * ROW,
                                         CPB * ROW)],
                            rows.at[b, pl.ds(0, CPB * ROW)])
            for j in range(CPB):
                pltpu.async_copy(rows.at[b, pl.ds(j * ROW, ROW)],
                                 z4.at[idxz_v.at[blk * CPB + j]], sem_s)
            return 0
        lax.fori_loop(0, npt // ROW // CPB, cblk, 0)
        for r in range(2):
            pltpu.make_async_copy(y4.at[pl.ds(0, CPB * ROW)],
                                  rows.at[r, pl.ds(0, CPB * ROW)],
                                  sem_s).wait()
        plsc.subcore_barrier()

    for qq in range(2):
        @pl.when(c == 0)
        def _(qq=qq):
            phase(qq)

        @pl.when(c == 1)
        def _(qq=qq):
            phase(2 + qq)


def _stage_a_body(comb, deg0, deg1, gate_wT, gate_b, w0, rw1T, rb1, rw2T,
                  rb2a, gate_o, y_o, ext_o):
    x = comb[...]
    deg = deg0[:, 0:1] + deg1[:, 0:1] + 1.0
    dinv = lax.rsqrt(deg)
    g = jax.nn.sigmoid(jnp.dot(x, gate_wT[...],
                               preferred_element_type=f32) + gate_b[...])
    y = dinv * jnp.dot(x, w0[...], preferred_element_type=f32)
    r = jax.nn.relu(jnp.dot(x, rw1T[...], preferred_element_type=f32)
                    + rb1[...])
    r = jnp.dot(r, rw2T[...], preferred_element_type=f32) + rb2a[0, 0:1]
    gate_o[...] = g
    y_o[...] = y
    ext_o[...] = jnp.concatenate(
        [dinv, r * rb2a[0, 1:2], jnp.zeros((BLK, H - 2), f32)], axis=-1)


def _rowspec(k):
    return pl.BlockSpec((BLK, k), lambda i: (i, 0))


def _wspec(r, k):
    return pl.BlockSpec((r, k), lambda i: (0, 0))


def _stage_a(comb, deg0, deg1, gate_wT, gate_b, w0, rw1T, rb1, rw2T, rb2a):
    return pl.pallas_call(
        _stage_a_body,
        grid=(GRID,),
        in_specs=[_rowspec(H), _rowspec(16), _rowspec(16), _wspec(H, H),
                  _wspec(1, H), _wspec(H, H), _wspec(H, 32), _wspec(1, 32),
                  _wspec(32, 1), _wspec(1, 2)],
        out_specs=[_rowspec(H), _rowspec(H), _rowspec(H)],
        out_shape=[jax.ShapeDtypeStruct((N, H), f32) for _ in range(3)],
    )(comb, deg0, deg1, gate_wT, gate_b, w0, rw1T, rb1, rw2T, rb2a)


def _stage_b_body(mode, z, y, ext, aux, b_c, ln_g, ln_b, w_n, h_o, y_o):
    di = ext[:, 0:1]
    t = di * (z[...] + y[...]) + b_c[...]
    m = jnp.mean(t, -1, keepdims=True)
    v = jnp.mean(t * t, -1, keepdims=True) - m * m
    t = (t - m) * lax.rsqrt(v + 1e-5) * ln_g[...] + ln_b[...]
    hn = jax.nn.relu(t)
    if mode == "gate":
        h = hn * aux[...]
    else:
        h = hn + aux[...]
    y_o[...] = di * jnp.dot(h, w_n[...], preferred_element_type=f32)
    h_o[...] = h


def _stage_b(mode, z, y, ext, aux, b_c, ln_g, ln_b, w_n):
    return pl.pallas_call(
        functools.partial(_stage_b_body, mode),
        grid=(GRID,),
        in_specs=[_rowspec(H), _rowspec(H), _rowspec(H), _rowspec(H),
                  _wspec(1, H), _wspec(1, H), _wspec(1, H), _wspec(H, H)],
        out_specs=[_rowspec(H), _rowspec(H)],
        out_shape=[jax.ShapeDtypeStruct((N, H), f32) for _ in range(2)],
    )(z, y, ext, aux, b_c, ln_g, ln_b, w_n)


def _stage_c_body(z, y, ext, hprev, b_c, ln_g, ln_b, pre_wT, pre_b, A,
                  post_b, out_o):
    di = ext[:, 0:1]
    t = di * (z[...] + y[...]) + b_c[...]
    m = jnp.mean(t, -1, keepdims=True)
    v = jnp.mean(t * t, -1, keepdims=True) - m * m
    t = (t - m) * lax.rsqrt(v + 1e-5) * ln_g[...] + ln_b[...]
    h = jax.nn.relu(t) + hprev[...]
    a = jnp.tanh(jnp.dot(h, pre_wT[...], preferred_element_type=f32)
                 + pre_b[...])
    cc = jnp.cos(a * 0.5)
    ss = jnp.sin(a * 0.5)
    cols = lax.broadcasted_iota(jnp.int32, (1, 16), 1)
    psi = jnp.ones((BLK, 16), f32)
    for q in range(4):
        bit = ((cols >> (3 - q)) & 1) == 1
        psi = psi * jnp.where(bit, ss[:, q:q + 1], cc[:, q:q + 1])
    pd = jnp.sum(jnp.dot(psi, A[...], preferred_element_type=f32) * psi,
                 -1, keepdims=True) + post_b[0, 0]
    out_o[...] = pd + ext[:, 1:2]


def _stage_c(z, y, ext, hprev, b_c, ln_g, ln_b, pre_wT, pre_b, A, post_b):
    return pl.pallas_call(
        _stage_c_body,
        grid=(GRID,),
        in_specs=[_rowspec(H), _rowspec(H), _rowspec(H), _rowspec(H),
                  _wspec(1, H), _wspec(1, H), _wspec(1, H), _wspec(H, 4),
                  _wspec(1, 4), _wspec(16, 16), _wspec(1, 1)],
        out_specs=[_rowspec(1)],
        out_shape=[jax.ShapeDtypeStruct((N, 1), f32)],
    )(z, y, ext, hprev, b_c, ln_g, ln_b, pre_wT, pre_b, A, post_b)


def _z_diags_np():
    b = np.arange(16)
    return np.stack([1.0 - 2.0 * ((b >> (3 - i)) & 1)
                     for i in range(4)]).astype(np.float32)


def _cnot_np(c, t):
    M = np.zeros((16, 16), dtype=np.complex64)
    for b in range(16):
        b2 = b ^ (1 << (3 - t)) if (b >> (3 - c)) & 1 else b
        M[b2, b] = 1.0
    return jnp.asarray(M)


def _rot_j(phi, theta, omega):
    em = jnp.exp(-0.5j * phi).astype(jnp.complex64)
    ep = jnp.exp(0.5j * phi).astype(jnp.complex64)
    z = jnp.zeros((), jnp.complex64)
    rz1 = jnp.stack([jnp.stack([em, z]), jnp.stack([z, ep])])
    cth = jnp.cos(theta / 2).astype(jnp.complex64)
    sth = jnp.sin(theta / 2).astype(jnp.complex64)
    ry = jnp.stack([jnp.stack([cth, -sth]), jnp.stack([sth, cth])])
    em2 = jnp.exp(-0.5j * omega).astype(jnp.complex64)
    ep2 = jnp.exp(0.5j * omega).astype(jnp.complex64)
    rz2 = jnp.stack([jnp.stack([em2, z]), jnp.stack([z, ep2])])
    return rz2 @ ry @ rz1


def _quad_form(q_weights, post_w):
    U = jnp.eye(16, dtype=jnp.complex64)
    for l in range(q_weights.shape[0]):
        R = _rot_j(q_weights[l, 0, 0], q_weights[l, 0, 1], q_weights[l, 0, 2])
        for q in range(1, 4):
            R = jnp.kron(R, _rot_j(q_weights[l, q, 0], q_weights[l, q, 1],
                                   q_weights[l, q, 2]))
        U = R @ U
        r = (l % 3) + 1
        for i in range(4):
            U = _cnot_np(i, (i + r) % 4) @ U
    g = post_w[0] @ jnp.asarray(_z_diags_np())
    return jnp.real(jnp.conj(U.T) @ (g[:, None] * U))


def kernel(x, pk_embeddings, pk_predictions, edge_index, gate_w, gate_b,
           conv_w0, conv_b0, conv_w1, conv_b1, conv_w2, conv_b2,
           ln_g0, ln_b0, ln_g1, ln_b1, ln_g2, ln_b2,
           pre_w, pre_b, q_weights, post_w, post_b,
           res_w1, res_b1, res_w2, res_b2, res_alpha):
    comb = jnp.concatenate([x, pk_embeddings, pk_predictions], axis=-1)
    pad = EPAD - E
    src2d = jnp.concatenate(
        [edge_index[0], jnp.zeros((pad,), jnp.int32)]).reshape(NROWS, ROW)
    dst2d = jnp.concatenate(
        [edge_index[1], jnp.full((pad,), NPAD, jnp.int32)]).reshape(NROWS,
                                                                    ROW)
    src4 = (4 * src2d)[None, :, :] + jnp.arange(4, dtype=jnp.int32).reshape(
        4, 1, 1)
    node = jnp.minimum(
        jnp.arange(16, dtype=jnp.int32).reshape(16, 1, 1) * (NPAD // 16)
        + jnp.arange(32, dtype=jnp.int32).reshape(1, 32, 1) * ROW
        + jnp.arange(ROW, dtype=jnp.int32).reshape(1, 1, ROW), NPAD - 1)
    idxz = (4 * node)[None] + jnp.arange(4, dtype=jnp.int32).reshape(
        4, 1, 1, 1)

    deg0, deg1 = _deg_kernel(dst2d)

    rb2a = jnp.stack([res_b2[0], res_alpha]).reshape(1, 2)
    gate, y64, ext = _stage_a(
        comb, deg0, deg1, gate_w.T, gate_b.reshape(1, H), conv_w0,
        res_w1.T, res_b1.reshape(1, 32), res_w2.T, rb2a)

    z64 = _scatter_kernel(src4, dst2d, idxz,
                          y64.reshape(4 * N, HH)).reshape(NPAD, H)
    h1, y64 = _stage_b("gate", z64, y64, ext, gate,
                       conv_b0.reshape(1, H), ln_g0.reshape(1, H),
                       ln_b0.reshape(1, H), conv_w1)

    z64 = _scatter_kernel(src4, dst2d, idxz,
                          y64.reshape(4 * N, HH)).reshape(NPAD, H)
    h2, y64 = _stage_b("res", z64, y64, ext, h1,
                       conv_b1.reshape(1, H), ln_g1.reshape(1, H),
                       ln_b1.reshape(1, H), conv_w2)

    z64 = _scatter_kernel(src4, dst2d, idxz,
                          y64.reshape(4 * N, HH)).reshape(NPAD, H)
    A = _quad_form(q_weights, post_w)
    (out,) = _stage_c(z64, y64, ext, h2,
                      conv_b2.reshape(1, H), ln_g2.reshape(1, H),
                      ln_b2.reshape(1, H), pre_w.T, pre_b.reshape(1, 4),
                      A, post_b.reshape(1, 1))
    return out


# speedup vs baseline: 13.2320x; 1.0163x over previous
import functools
import numpy as np
import jax
import jax.numpy as jnp
from jax import lax
from jax.experimental import pallas as pl
from jax.experimental.pallas import tpu as pltpu
from jax.experimental.pallas import tpu_sc as plsc

N = 50000
NPAD = 51200
ATOT = NPAD + 128
E = 800000
EPAD = 819200
H = 64
HH = 16
NS = 4
ROW = 128
NROWS = EPAD // ROW
GROUP = 8
SGROUP = 10
ZCH = 128
BLK = 1000
GRID = N // BLK

_mesh = plsc.VectorSubcoreMesh(core_axis_name="c", subcore_axis_name="s")
f32 = jnp.float32
_sc_params = pltpu.CompilerParams(use_tc_tiling_on_sc=False)


@functools.partial(
    pl.kernel,
    out_type=[jax.ShapeDtypeStruct((NPAD, 16), f32),
              jax.ShapeDtypeStruct((NPAD, 16), f32)],
    mesh=_mesh,
    scratch_types=[
        pltpu.VMEM((ROW, 16), f32),
        pltpu.VMEM((ZCH, 16), f32),
        pltpu.VMEM((GROUP, ROW), jnp.int32),
        pltpu.VMEM_SHARED((ATOT, 16), f32),
    ],
    compiler_params=_sc_params,
)
def _deg_kernel(dst2d, deg0_out, deg1_out, ones_v, zero_v, idx_d, acc):
    c = lax.axis_index("c")
    s = lax.axis_index("s")
    npt = NPAD // 16

    def fill(i, _):
        ones_v[i, :] = jnp.ones((16,), f32)
        return 0
    lax.fori_loop(0, ROW, fill, 0)

    def fillz(i, _):
        zero_v[i, :] = jnp.zeros((16,), f32)
        return 0
    lax.fori_loop(0, ZCH, fillz, 0)

    def zero(i, _):
        pltpu.sync_copy(zero_v, acc.at[pl.ds(s * npt + i * ZCH, ZCH)])
        return 0
    lax.fori_loop(0, npt // ZCH, zero, 0)
    plsc.subcore_barrier()

    rows_per_tile = NROWS // 32
    base = (c * 16 + s) * rows_per_tile

    def grp(g, _):
        pltpu.sync_copy(dst2d.at[pl.ds(base + g * GROUP, GROUP)], idx_d)
        for j in range(GROUP):
            pltpu.sync_copy(ones_v, acc.at[idx_d.at[j]], add=True)
        return 0
    lax.fori_loop(0, rows_per_tile // GROUP, grp, 0)
    plsc.subcore_barrier()

    @pl.when(c == 0)
    def _():
        pltpu.sync_copy(acc.at[pl.ds(s * npt, npt)],
                        deg0_out.at[pl.ds(s * npt, npt)])

    @pl.when(c == 1)
    def _():
        pltpu.sync_copy(acc.at[pl.ds(s * npt, npt)],
                        deg1_out.at[pl.ds(s * npt, npt)])


@functools.partial(
    pl.kernel,
    out_type=jax.ShapeDtypeStruct((4 * NPAD, HH), f32),
    mesh=_mesh,
    scratch_types=[
        pltpu.VMEM((3, SGROUP, ROW), jnp.int32),
        pltpu.VMEM((3, SGROUP, ROW), jnp.int32),
        pltpu.VMEM((3, SGROUP * ROW, HH), f32),
        pltpu.VMEM((32, ROW), jnp.int32),
        pltpu.VMEM((ZCH, HH), f32),
        pltpu.VMEM_SHARED((ATOT, HH), f32),
        pltpu.SemaphoreType.DMA,
        pltpu.SemaphoreType.DMA,
    ],
    compiler_params=_sc_params,
)
def _scatter_kernel(src4, dst2d, idxz, y4, z4,
                    idx_s, idx_d, rows, idxz_v, zero_v, acc, sem_g, sem_s):
    c = lax.axis_index("c")
    s = lax.axis_index("s")
    npt = NPAD // 16

    def fillz(i, _):
        zero_v[i, :] = jnp.zeros((16,), f32)
        return 0
    lax.fori_loop(0, ZCH, fillz, 0)

    rows_per_tile = NROWS // 16
    ngrp = rows_per_tile // SGROUP
    base = s * rows_per_tile

    def phase(q):
        def zero(i, _):
            pltpu.sync_copy(zero_v, acc.at[pl.ds(s * npt + i * ZCH, ZCH)])
            return 0
        lax.fori_loop(0, npt // ZCH, zero, 0)
        plsc.subcore_barrier()

        def load_and_fire(g, b):
            r0 = base + g * SGROUP
            pltpu.sync_copy(src4.at[q, pl.ds(r0, SGROUP)], idx_s.at[b])
            pltpu.sync_copy(dst2d.at[pl.ds(r0, SGROUP)], idx_d.at[b])
            for j in range(SGROUP):
                pltpu.async_copy(y4.at[idx_s.at[b, j]],
                                 rows.at[b, pl.ds(j * ROW, ROW)], sem_g)

        load_and_fire(0, 0)

        def grp(g, _):
            p = lax.rem(g, 3)
            pn = lax.rem(g + 1, 3)

            @pl.when(g + 1 < ngrp)
            def _():
                @pl.when(g >= 2)
                def _():
                    pltpu.make_async_copy(y4.at[pl.ds(0, SGROUP * ROW)],
                                          rows.at[pn], sem_s).wait()
                load_and_fire(g + 1, pn)

            pltpu.make_async_copy(y4.at[pl.ds(0, SGROUP * ROW)],
                                  rows.at[p], sem_g).wait()
            for j in range(SGROUP):
                pltpu.async_copy(rows.at[p, pl.ds(j * ROW, ROW)],
                                 acc.at[idx_d.at[p, j]], sem_s, add=True)
            return 0
        lax.fori_loop(0, ngrp, grp, 0)
        for r in range(3):
            pltpu.make_async_copy(y4.at[pl.ds(0, SGROUP * ROW)],
                                  rows.at[r], sem_s).wait()
        plsc.subcore_barrier()
        CPB = 5
        pltpu.sync_copy(idxz.at[q, s], idxz_v)

        def cblk(blk, _):
            b = lax.rem(blk, 2)

            @pl.when(blk >= 2)
            def _():
                pltpu.make_async_copy(y4.at[pl.ds(0, CPB * ROW)],
                                      rows.at[2, pl.ds(0, CPB * ROW)],
                                      sem_s).wait()
            pltpu.sync_copy(acc.at[pl.ds(s * npt + blk * CPB * ROW,
                                         CPB * ROW)],
                            rows.at[b, pl.ds(0, CPB * ROW)])
            for j in range(CPB):
                pltpu.async_copy(rows.at[b, pl.ds(j * ROW, ROW)],
                                 z4.at[idxz_v.at[blk * CPB + j]], sem_s)
            return 0
        lax.fori_loop(0, npt // ROW // CPB, cblk, 0)
        for r in range(2):
            pltpu.make_async_copy(y4.at[pl.ds(0, CPB * ROW)],
                                  rows.at[r, pl.ds(0, CPB * ROW)],
                                  sem_s).wait()
        plsc.subcore_barrier()

    for qq in range(2):
        @pl.when(c == 0)
        def _(qq=qq):
            phase(qq)

        @pl.when(c == 1)
        def _(qq=qq):
            phase(2 + qq)


def _stage_a_body(comb, deg0, deg1, gate_wT, gate_b, w0, rw1T, rb1, rw2T,
                  rb2a, gate_o, y_o, ext_o):
    x = comb[...]
    deg = deg0[:, 0:1] + deg1[:, 0:1] + 1.0
    dinv = lax.rsqrt(deg)
    g = jax.nn.sigmoid(jnp.dot(x, gate_wT[...],
                               preferred_element_type=f32) + gate_b[...])
    y = dinv * jnp.dot(x, w0[...], preferred_element_type=f32)
    r = jax.nn.relu(jnp.dot(x, rw1T[...], preferred_element_type=f32)
                    + rb1[...])
    r = jnp.dot(r, rw2T[...], preferred_element_type=f32) + rb2a[0, 0:1]
    gate_o[...] = g
    y_o[...] = y
    ext_o[...] = jnp.concatenate(
        [dinv, r * rb2a[0, 1:2], jnp.zeros((BLK, H - 2), f32)], axis=-1)


def _rowspec(k):
    return pl.BlockSpec((BLK, k), lambda i: (i, 0))


def _wspec(r, k):
    return pl.BlockSpec((r, k), lambda i: (0, 0))


def _stage_a(comb, deg0, deg1, gate_wT, gate_b, w0, rw1T, rb1, rw2T, rb2a):
    return pl.pallas_call(
        _stage_a_body,
        grid=(GRID,),
        in_specs=[_rowspec(H), _rowspec(16), _rowspec(16), _wspec(H, H),
                  _wspec(1, H), _wspec(H, H), _wspec(H, 32), _wspec(1, 32),
                  _wspec(32, 1), _wspec(1, 2)],
        out_specs=[_rowspec(H), _rowspec(H), _rowspec(H)],
        out_shape=[jax.ShapeDtypeStruct((N, H), f32) for _ in range(3)],
    )(comb, deg0, deg1, gate_wT, gate_b, w0, rw1T, rb1, rw2T, rb2a)


def _stage_b_body(mode, z, y, ext, aux, b_c, ln_g, ln_b, w_n, h_o, y_o):
    di = ext[:, 0:1]
    t = di * (z[...] + y[...]) + b_c[...]
    m = jnp.mean(t, -1, keepdims=True)
    v = jnp.mean(t * t, -1, keepdims=True) - m * m
    t = (t - m) * lax.rsqrt(v + 1e-5) * ln_g[...] + ln_b[...]
    hn = jax.nn.relu(t)
    if mode == "gate":
        h = hn * aux[...]
    else:
        h = hn + aux[...]
    y_o[...] = di * jnp.dot(h, w_n[...], preferred_element_type=f32)
    h_o[...] = h


def _stage_b(mode, z, y, ext, aux, b_c, ln_g, ln_b, w_n):
    return pl.pallas_call(
        functools.partial(_stage_b_body, mode),
        grid=(GRID,),
        in_specs=[_rowspec(H), _rowspec(H), _rowspec(H), _rowspec(H),
                  _wspec(1, H), _wspec(1, H), _wspec(1, H), _wspec(H, H)],
        out_specs=[_rowspec(H), _rowspec(H)],
        out_shape=[jax.ShapeDtypeStruct((N, H), f32) for _ in range(2)],
    )(z, y, ext, aux, b_c, ln_g, ln_b, w_n)


def _stage_c_body(z, y, ext, hprev, b_c, ln_g, ln_b, pre_wT, pre_b, A,
                  post_b, out_o):
    di = ext[:, 0:1]
    t = di * (z[...] + y[...]) + b_c[...]
    m = jnp.mean(t, -1, keepdims=True)
    v = jnp.mean(t * t, -1, keepdims=True) - m * m
    t = (t - m) * lax.rsqrt(v + 1e-5) * ln_g[...] + ln_b[...]
    h = jax.nn.relu(t) + hprev[...]
    a = jnp.tanh(jnp.dot(h, pre_wT[...], preferred_element_type=f32)
                 + pre_b[...])
    cc = jnp.cos(a * 0.5)
    ss = jnp.sin(a * 0.5)
    cols = lax.broadcasted_iota(jnp.int32, (1, 16), 1)
    psi = jnp.ones((BLK, 16), f32)
    for q in range(4):
        bit = ((cols >> (3 - q)) & 1) == 1
        psi = psi * jnp.where(bit, ss[:, q:q + 1], cc[:, q:q + 1])
    pd = jnp.sum(jnp.dot(psi, A[...], preferred_element_type=f32) * psi,
                 -1, keepdims=True) + post_b[0, 0]
    out_o[...] = pd + ext[:, 1:2]


def _stage_c(z, y, ext, hprev, b_c, ln_g, ln_b, pre_wT, pre_b, A, post_b):
    return pl.pallas_call(
        _stage_c_body,
        grid=(GRID,),
        in_specs=[_rowspec(H), _rowspec(H), _rowspec(H), _rowspec(H),
                  _wspec(1, H), _wspec(1, H), _wspec(1, H), _wspec(H, 4),
                  _wspec(1, 4), _wspec(16, 16), _wspec(1, 1)],
        out_specs=[_rowspec(1)],
        out_shape=[jax.ShapeDtypeStruct((N, 1), f32)],
    )(z, y, ext, hprev, b_c, ln_g, ln_b, pre_wT, pre_b, A, post_b)


def _z_diags_np():
    b = np.arange(16)
    return np.stack([1.0 - 2.0 * ((b >> (3 - i)) & 1)
                     for i in range(4)]).astype(np.float32)


def _cnot_np(c, t):
    M = np.zeros((16, 16), dtype=np.complex64)
    for b in range(16):
        b2 = b ^ (1 << (3 - t)) if (b >> (3 - c)) & 1 else b
        M[b2, b] = 1.0
    return jnp.asarray(M)


def _rot_j(phi, theta, omega):
    em = jnp.exp(-0.5j * phi).astype(jnp.complex64)
    ep = jnp.exp(0.5j * phi).astype(jnp.complex64)
    z = jnp.zeros((), jnp.complex64)
    rz1 = jnp.stack([jnp.stack([em, z]), jnp.stack([z, ep])])
    cth = jnp.cos(theta / 2).astype(jnp.complex64)
    sth = jnp.sin(theta / 2).astype(jnp.complex64)
    ry = jnp.stack([jnp.stack([cth, -sth]), jnp.stack([sth, cth])])
    em2 = jnp.exp(-0.5j * omega).astype(jnp.complex64)
    ep2 = jnp.exp(0.5j * omega).astype(jnp.complex64)
    rz2 = jnp.stack([jnp.stack([em2, z]), jnp.stack([z, ep2])])
    return rz2 @ ry @ rz1


def _quad_form(q_weights, post_w):
    U = jnp.eye(16, dtype=jnp.complex64)
    for l in range(q_weights.shape[0]):
        R = _rot_j(q_weights[l, 0, 0], q_weights[l, 0, 1], q_weights[l, 0, 2])
        for q in range(1, 4):
            R = jnp.kron(R, _rot_j(q_weights[l, q, 0], q_weights[l, q, 1],
                                   q_weights[l, q, 2]))
        U = R @ U
        r = (l % 3) + 1
        for i in range(4):
            U = _cnot_np(i, (i + r) % 4) @ U
    g = post_w[0] @ jnp.asarray(_z_diags_np())
    return jnp.real(jnp.conj(U.T) @ (g[:, None] * U))


def kernel(x, pk_embeddings, pk_predictions, edge_index, gate_w, gate_b,
           conv_w0, conv_b0, conv_w1, conv_b1, conv_w2, conv_b2,
           ln_g0, ln_b0, ln_g1, ln_b1, ln_g2, ln_b2,
           pre_w, pre_b, q_weights, post_w, post_b,
           res_w1, res_b1, res_w2, res_b2, res_alpha):
    comb = jnp.concatenate([x, pk_embeddings, pk_predictions], axis=-1)
    pad = EPAD - E
    src2d = jnp.concatenate(
        [edge_index[0], jnp.zeros((pad,), jnp.int32)]).reshape(NROWS, ROW)
    dst2d = jnp.concatenate(
        [edge_index[1], jnp.full((pad,), NPAD, jnp.int32)]).reshape(NROWS,
                                                                    ROW)
    src4 = (4 * src2d)[None, :, :] + jnp.arange(4, dtype=jnp.int32).reshape(
        4, 1, 1)
    node = jnp.minimum(
        jnp.arange(16, dtype=jnp.int32).reshape(16, 1, 1) * (NPAD // 16)
        + jnp.arange(32, dtype=jnp.int32).reshape(1, 32, 1) * ROW
        + jnp.arange(ROW, dtype=jnp.int32).reshape(1, 1, ROW), NPAD - 1)
    idxz = (4 * node)[None] + jnp.arange(4, dtype=jnp.int32).reshape(
        4, 1, 1, 1)

    deg0, deg1 = _deg_kernel(dst2d)

    rb2a = jnp.stack([res_b2[0], res_alpha]).reshape(1, 2)
    gate, y64, ext = _stage_a(
        comb, deg0, deg1, gate_w.T, gate_b.reshape(1, H), conv_w0,
        res_w1.T, res_b1.reshape(1, 32), res_w2.T, rb2a)

    z64 = _scatter_kernel(src4, dst2d, idxz,
                          y64.reshape(4 * N, HH)).reshape(NPAD, H)
    h1, y64 = _stage_b("gate", z64, y64, ext, gate,
                       conv_b0.reshape(1, H), ln_g0.reshape(1, H),
                       ln_b0.reshape(1, H), conv_w1)

    z64 = _scatter_kernel(src4, dst2d, idxz,
                          y64.reshape(4 * N, HH)).reshape(NPAD, H)
    h2, y64 = _stage_b("res", z64, y64, ext, h1,
                       conv_b1.reshape(1, H), ln_g1.reshape(1, H),
                       ln_b1.reshape(1, H), conv_w2)

    z64 = _scatter_kernel(src4, dst2d, idxz,
                          y64.reshape(4 * N, HH)).reshape(NPAD, H)
    A = _quad_form(q_weights, post_w)
    (out,) = _stage_c(z64, y64, ext, h2,
                      conv_b2.reshape(1, H), ln_g2.reshape(1, H),
                      ln_b2.reshape(1, H), pre_w.T, pre_b.reshape(1, 4),
                      A, post_b.reshape(1, 1))
    return out
